# baseline (speedup 1.0000x reference)
import functools

import jax
import jax.numpy as jnp
from jax import lax
from jax.experimental import pallas as pl
from jax.experimental.pallas import tpu as pltpu
from jax.experimental.pallas import tpu_sc as plsc

_BE = 3200
_EPS = 1e-5

_N = 10000
_E = 320000
_D = 128
_NC = 1
_NS = 16
_NW = _NC * _NS
_EPW = _E // _NW
_K = 40
_NCHUNK = _EPW // _K
_RPS = 624
_ZR = 104


def _edge_tc_body(first, has_u, last, *refs):
    i = 0
    if first:
        dist_ref = refs[i]; i += 1
        wedge_ref = refs[i]; i += 1
        bedge_ref = refs[i]; i += 1
    else:
        wprev_ref = refs[i]; i += 1
    if has_u:
        u_ref = refs[i]; i += 1
        a_ref = refs[i]; i += 1
        b_ref = refs[i]; i += 1
    if not last:
        we_ref = refs[i]; i += 1
        be_ref = refs[i]; i += 1
    wout_ref = refs[i]; i += 1
    if not last:
        tout_ref = refs[i]; i += 1

    if first:
        w_prev = dist_ref[...] * wedge_ref[...] + bedge_ref[...]
    else:
        w_prev = wprev_ref[...]
    if has_u:
        z = u_ref[...] * a_ref[...] + b_ref[...]
        w = w_prev + z * jax.nn.sigmoid(z)
    else:
        w = w_prev
    wout_ref[...] = w
    if not last:
        tout_ref[...] = (
            jnp.dot(w, we_ref[...], preferred_element_type=jnp.float32)
            + be_ref[...]
        )


def _edge_tc(first, has_u, last, *args, E, D):
    grid = (E // _BE,)
    row_spec = pl.BlockSpec((_BE, D), lambda i: (i, 0))
    dist_spec = pl.BlockSpec((_BE, 1), lambda i: (i, 0))
    vec_spec = pl.BlockSpec((1, D), lambda i: (0, 0))
    mat_spec = pl.BlockSpec((D, D), lambda i: (0, 0))
    in_specs = []
    if first:
        in_specs += [dist_spec, vec_spec, vec_spec]
    else:
        in_specs += [row_spec]
    if has_u:
        in_specs += [row_spec, vec_spec, vec_spec]
    if not last:
        in_specs += [mat_spec, vec_spec]
    out_specs = [row_spec] + ([] if last else [row_spec])
    out_shape = [jax.ShapeDtypeStruct((E, D), jnp.float32)]
    if not last:
        out_shape.append(jax.ShapeDtypeStruct((E, D), jnp.float32))
    return pl.pallas_call(
        functools.partial(_edge_tc_body, first, has_u, last),
        grid=grid,
        in_specs=in_specs,
        out_specs=out_specs,
        out_shape=out_shape,
    )(*args)


def _node_body(nparts, lastlayer, *refs):
    i = 0
    x_ref = refs[i]; i += 1
    x1_ref = refs[i]; i += 1
    aggp_ref = refs[i]; i += 1
    counts_ref = refs[i]; i += 1
    gamma_ref = refs[i]; i += 1
    beta_ref = refs[i]; i += 1
    if not lastlayer:
        wv_ref = refs[i]; i += 1
        bv_ref = refs[i]; i += 1
    xout_ref = refs[i]; i += 1
    if not lastlayer:
        x1o_ref = refs[i]; i += 1
        x24o_ref = refs[i]; i += 1
        x3o_ref = refs[i]; i += 1

    agg = aggp_ref[0]
    for p in range(1, nparts):
        agg = agg + aggp_ref[p]
    agg = agg / counts_ref[...]
    pre = x1_ref[...] + agg
    mean = jnp.mean(pre, axis=0, keepdims=True)
    var = jnp.mean((pre - mean) ** 2, axis=0, keepdims=True)
    xn = gamma_ref[...] * (pre - mean) / jnp.sqrt(var + _EPS) + beta_ref[...]
    x = x_ref[...] + xn * jax.nn.sigmoid(xn)
    xout_ref[...] = x
    if not lastlayer:
        D = x.shape[1]
        mm = [
            jnp.dot(x, wv_ref[k], preferred_element_type=jnp.float32)
            + bv_ref[0, k][None, :]
            for k in range(4)
        ]
        x1o_ref[...] = mm[0]
        x24o_ref[:, :D] = mm[1]
        x24o_ref[:, D:] = mm[3]
        x3o_ref[...] = mm[2]


def _node_tc(x, x1, aggp, counts, gamma, beta, wv, bv, lastlayer, N, D):
    nparts = aggp.shape[0]
    args = [x, x1, aggp, counts, gamma.reshape(1, D), beta.reshape(1, D)]
    in_specs = [
        pl.BlockSpec((N, D), lambda: (0, 0)),
        pl.BlockSpec((N, D), lambda: (0, 0)),
        pl.BlockSpec((nparts, N, D), lambda: (0, 0, 0)),
        pl.BlockSpec((N, 1), lambda: (0, 0)),
        pl.BlockSpec((1, D), lambda: (0, 0)),
        pl.BlockSpec((1, D), lambda: (0, 0)),
    ]
    out_shape = [jax.ShapeDtypeStruct((N, D), jnp.float32)]
    out_specs = [pl.BlockSpec((N, D), lambda: (0, 0))]
    if not lastlayer:
        args += [wv, bv.reshape(1, 4, D)]
        in_specs += [
            pl.BlockSpec((4, D, D), lambda: (0, 0, 0)),
            pl.BlockSpec((1, 4, D), lambda: (0, 0, 0)),
        ]
        out_shape += [
            jax.ShapeDtypeStruct((N, D), jnp.float32),
            jax.ShapeDtypeStruct((N, 2 * D), jnp.float32),
            jax.ShapeDtypeStruct((N, D), jnp.float32),
        ]
        out_specs += [
            pl.BlockSpec((N, D), lambda: (0, 0)),
            pl.BlockSpec((N, 2 * D), lambda: (0, 0)),
            pl.BlockSpec((N, D), lambda: (0, 0)),
        ]
    return pl.pallas_call(
        functools.partial(_node_body, nparts, lastlayer),
        in_specs=in_specs,
        out_specs=out_specs,
        out_shape=out_shape,
    )(*args)


def _init_body(locs_ref, wi_ref, bi_ref, wv_ref, bv_ref,
               ne_ref, x1o_ref, x24o_ref, x3o_ref):
    ne = (
        jnp.dot(locs_ref[...], wi_ref[...], preferred_element_type=jnp.float32)
        + bi_ref[...]
    )
    ne_ref[...] = ne
    D = ne.shape[1]
    mm = [
        jnp.dot(ne, wv_ref[k], preferred_element_type=jnp.float32)
        + bv_ref[0, k][None, :]
        for k in range(4)
    ]
    x1o_ref[...] = mm[0]
    x24o_ref[:, :D] = mm[1]
    x24o_ref[:, D:] = mm[3]
    x3o_ref[...] = mm[2]


def _init_tc(locs, W_init, b_init, wv0, bv0, N, D):
    return pl.pallas_call(
        _init_body,
        in_specs=[
            pl.BlockSpec((N, 2), lambda: (0, 0)),
            pl.BlockSpec((2, D), lambda: (0, 0)),
            pl.BlockSpec((1, D), lambda: (0, 0)),
            pl.BlockSpec((4, D, D), lambda: (0, 0, 0)),
            pl.BlockSpec((1, 4, D), lambda: (0, 0, 0)),
        ],
        out_specs=[
            pl.BlockSpec((N, D), lambda: (0, 0)),
            pl.BlockSpec((N, D), lambda: (0, 0)),
            pl.BlockSpec((N, 2 * D), lambda: (0, 0)),
            pl.BlockSpec((N, D), lambda: (0, 0)),
        ],
        out_shape=[
            jax.ShapeDtypeStruct((N, D), jnp.float32),
            jax.ShapeDtypeStruct((N, D), jnp.float32),
            jax.ShapeDtypeStruct((N, 2 * D), jnp.float32),
            jax.ShapeDtypeStruct((N, D), jnp.float32),
        ],
    )(locs, W_init, b_init.reshape(1, D), wv0, bv0.reshape(1, 4, D))


def _sc_edge_body(src_hbm, dst_hbm, w_hbm, t_hbm, x24_hbm, x3_hbm,
                  u_hbm, aggp_hbm, stats_hbm,
                  srcv, dstv, wv, tv, g24v, g3v, msgsv, ustg, statsv, zbuf,
                  aggsh, s1, s2, s3, s4):
    D = _D
    cid = lax.axis_index("c")
    sid = lax.axis_index("s")
    wid = sid * _NC + cid
    base_e = wid * _EPW
    zero = jnp.zeros((16,), jnp.float32)

    def zrow(j, carry):
        for c in range(D // 16):
            zbuf[j, pl.ds(c * 16, 16)] = zero
        return carry

    lax.fori_loop(0, 16, zrow, 0)
    for r in range(16):
        statsv[r, :] = zero
    row0 = sid * _RPS
    for i in range(_RPS // 16):
        pltpu.sync_copy(zbuf, aggsh.at[pl.ds(row0 + i * 16, 16)])

    @pl.when(sid == _NS - 1)
    def _zero_tail():
        pltpu.sync_copy(zbuf, aggsh.at[pl.ds(_NS * _RPS, 16)])

    plsc.subcore_barrier()

    def chunk(g, carry):
        b = base_e + g * _K
        pltpu.sync_copy(src_hbm.at[pl.ds(b, _K)], srcv)
        pltpu.sync_copy(dst_hbm.at[pl.ds(b, _K)], dstv)
        cp1 = pltpu.async_copy(w_hbm.at[pl.ds(b, _K)], wv, s1)
        cp2 = pltpu.async_copy(t_hbm.at[pl.ds(b, _K)], tv, s2)
        cp3 = pltpu.async_copy(x24_hbm.at[dstv], g24v, s3)
        cp4 = pltpu.async_copy(x3_hbm.at[srcv], g3v, s4)
        cp1.wait()
        cp2.wait()
        cp3.wait()
        cp4.wait()

        def row(j, rcarry):
            for c in range(D // 16):
                sl = pl.ds(c * 16, 16)
                wvec = wv[j, sl]
                sig = 1.0 / (1.0 + jnp.exp(-wvec))
                msgsv[j, sl] = sig * g24v[j, sl]
                uvec = tv[j, sl] + g3v[j, sl] + g24v[j, pl.ds(D + c * 16, 16)]
                ustg[j, sl] = uvec
                plsc.addupdate(statsv.at[c], uvec)
                plsc.addupdate(statsv.at[8 + c], uvec * uvec)
            return rcarry

        lax.fori_loop(0, _K, row, 0)
        pltpu.sync_copy(msgsv, aggsh.at[srcv], add=True)
        pltpu.sync_copy(ustg, u_hbm.at[pl.ds(b, _K)])
        return carry

    lax.fori_loop(0, _NCHUNK, chunk, 0)
    pltpu.sync_copy(statsv, stats_hbm.at[wid])
    plsc.subcore_barrier()
    for i in range(_RPS // _ZR):
        r = row0 + i * _ZR
        pltpu.sync_copy(aggsh.at[pl.ds(r, _ZR)], aggp_hbm.at[cid, pl.ds(r, _ZR)])

    @pl.when(sid == _NS - 1)
    def _flush_tail():
        r = _NS * _RPS
        pltpu.sync_copy(aggsh.at[pl.ds(r, 16)], aggp_hbm.at[cid, pl.ds(r, 16)])


_sc_edge = functools.partial(
    pl.kernel,
    out_type=[
        jax.ShapeDtypeStruct((_E, _D), jnp.float32),
        jax.ShapeDtypeStruct((_NC, _N, _D), jnp.float32),
        jax.ShapeDtypeStruct((_NW, 16, 16), jnp.float32),
    ],
    mesh=plsc.VectorSubcoreMesh(
        core_axis_name="c", subcore_axis_name="s", num_cores=_NC),
    scratch_types=[
        pltpu.VMEM((_K,), jnp.int32),
        pltpu.VMEM((_K,), jnp.int32),
        pltpu.VMEM((_K, _D), jnp.float32),
        pltpu.VMEM((_K, _D), jnp.float32),
        pltpu.VMEM((_K, 2 * _D), jnp.float32),
        pltpu.VMEM((_K, _D), jnp.float32),
        pltpu.VMEM((_K, _D), jnp.float32),
        pltpu.VMEM((_K, _D), jnp.float32),
        pltpu.VMEM((16, 16), jnp.float32),
        pltpu.VMEM((16, _D), jnp.float32),
        pltpu.VMEM_SHARED((_N, _D), jnp.float32),
        pltpu.SemaphoreType.DMA,
        pltpu.SemaphoreType.DMA,
        pltpu.SemaphoreType.DMA,
        pltpu.SemaphoreType.DMA,
    ],
)(_sc_edge_body)


def _bn_affine(s, ss, count, gamma, beta):
    mean = s / count
    var = ss / count - mean * mean
    inv = gamma / jnp.sqrt(var + _EPS)
    return inv, beta - mean * inv


def kernel(locs, edge_index, W_init, b_init, W_edge, b_edge, Wv, bv, We, be,
           gamma_v, beta_v, gamma_e, beta_e):
    N, D = locs.shape[0], W_init.shape[1]
    E = edge_index.shape[1]
    L = Wv.shape[0]
    src = edge_index[0]
    dst = edge_index[1]

    dl = locs[src] - locs[dst]
    dist = jnp.sqrt(dl[:, 0] ** 2 + dl[:, 1] ** 2 + 1e-12)
    ones = jnp.ones((E,), jnp.float32)
    counts = jnp.maximum(
        jax.ops.segment_sum(ones, src, num_segments=N), 1.0
    ).reshape(N, 1)

    dist2 = dist.reshape(E, 1)
    wedge = W_edge.reshape(1, D)
    bedge = b_edge.reshape(1, D)

    node_embed, x1, x24, x3 = _init_tc(locs, W_init, b_init, Wv[0], bv[0], N, D)
    x = node_embed

    w_prev = None
    u_prev = None
    su = ssu = None
    for l in range(L):
        if l == 0:
            w_cur, t = _edge_tc(
                True, False, False,
                dist2, wedge, bedge, We[0], be[0].reshape(1, D), E=E, D=D)
        elif l == 1:
            A, B = _bn_affine(su, ssu, float(E), gamma_e[l - 1], beta_e[l - 1])
            w_cur, t = _edge_tc(
                True, True, False,
                dist2, wedge, bedge, u_prev, A.reshape(1, D), B.reshape(1, D),
                We[l], be[l].reshape(1, D), E=E, D=D)
        else:
            A, B = _bn_affine(su, ssu, float(E), gamma_e[l - 1], beta_e[l - 1])
            w_cur, t = _edge_tc(
                False, True, False,
                w_prev, u_prev, A.reshape(1, D), B.reshape(1, D),
                We[l], be[l].reshape(1, D), E=E, D=D)

        u, aggp, stats = _sc_edge(src, dst, w_cur, t, x24, x3)
        st = stats.reshape(_NW, 2, D).sum(axis=0)
        su, ssu = st[0], st[1]

        lastlayer = l == L - 1
        if lastlayer:
            x = _node_tc(x, x1, aggp, counts, gamma_v[l], beta_v[l],
                         None, None, True, N, D)[0]
        else:
            x, x1, x24, x3 = _node_tc(
                x, x1, aggp, counts, gamma_v[l], beta_v[l],
                Wv[l + 1], bv[l + 1], False, N, D)
        w_prev = w_cur
        u_prev = u

    A, B = _bn_affine(su, ssu, float(E), gamma_e[L - 1], beta_e[L - 1])
    (w_final,) = _edge_tc(
        False, True, True,
        w_prev, u_prev, A.reshape(1, D), B.reshape(1, D), E=E, D=D)
    return (x, w_final, node_embed)


# speedup vs baseline: 1.3196x; 1.3196x over previous
import functools

import jax
import jax.numpy as jnp
from jax import lax
from jax.experimental import pallas as pl
from jax.experimental.pallas import tpu as pltpu
from jax.experimental.pallas import tpu_sc as plsc

_BE = 3200
_EPS = 1e-5

_N = 10000
_E = 320000
_D = 128
_NC = 1
_NS = 16
_NW = _NC * _NS
_EPW = _E // _NW
_K = 16
_NCHUNK = _EPW // _K
_NB = 4
_RPS = 624


def _edge_tc_body(first, has_u, last, *refs):
    i = 0
    if first:
        dist_ref = refs[i]; i += 1
        wedge_ref = refs[i]; i += 1
        bedge_ref = refs[i]; i += 1
    else:
        wprev_ref = refs[i]; i += 1
    if has_u:
        u_ref = refs[i]; i += 1
        a_ref = refs[i]; i += 1
        b_ref = refs[i]; i += 1
    if not last:
        we_ref = refs[i]; i += 1
        be_ref = refs[i]; i += 1
    wout_ref = refs[i]; i += 1
    if not last:
        tout_ref = refs[i]; i += 1

    if first:
        w_prev = dist_ref[...] * wedge_ref[...] + bedge_ref[...]
    else:
        w_prev = wprev_ref[...]
    if has_u:
        z = u_ref[...] * a_ref[...] + b_ref[...]
        w = w_prev + z * jax.nn.sigmoid(z)
    else:
        w = w_prev
    wout_ref[...] = w
    if not last:
        tout_ref[...] = (
            jnp.dot(w, we_ref[...], preferred_element_type=jnp.float32)
            + be_ref[...]
        )


def _edge_tc(first, has_u, last, *args, E, D):
    grid = (E // _BE,)
    row_spec = pl.BlockSpec((_BE, D), lambda i: (i, 0))
    dist_spec = pl.BlockSpec((_BE, 1), lambda i: (i, 0))
    vec_spec = pl.BlockSpec((1, D), lambda i: (0, 0))
    mat_spec = pl.BlockSpec((D, D), lambda i: (0, 0))
    in_specs = []
    if first:
        in_specs += [dist_spec, vec_spec, vec_spec]
    else:
        in_specs += [row_spec]
    if has_u:
        in_specs += [row_spec, vec_spec, vec_spec]
    if not last:
        in_specs += [mat_spec, vec_spec]
    out_specs = [row_spec] + ([] if last else [row_spec])
    out_shape = [jax.ShapeDtypeStruct((E, D), jnp.float32)]
    if not last:
        out_shape.append(jax.ShapeDtypeStruct((E, D), jnp.float32))
    return pl.pallas_call(
        functools.partial(_edge_tc_body, first, has_u, last),
        grid=grid,
        in_specs=in_specs,
        out_specs=out_specs,
        out_shape=out_shape,
    )(*args)


def _matmul_body(w_ref, we_ref, be_ref, t_ref):
    t_ref[...] = (
        jnp.dot(w_ref[...], we_ref[...], preferred_element_type=jnp.float32)
        + be_ref[...]
    )


def _matmul_tc(w, We, be, E, D):
    row_spec = pl.BlockSpec((_BE, D), lambda i: (i, 0))
    return pl.pallas_call(
        _matmul_body,
        grid=(E // _BE,),
        in_specs=[
            row_spec,
            pl.BlockSpec((D, D), lambda i: (0, 0)),
            pl.BlockSpec((1, D), lambda i: (0, 0)),
        ],
        out_specs=row_spec,
        out_shape=jax.ShapeDtypeStruct((E, D), jnp.float32),
    )(w, We, be)


def _stats_body(t_ref, g_ref, u_ref, st_ref):
    u = t_ref[...] + g_ref[...]
    u_ref[...] = u
    blk = jnp.concatenate(
        [jnp.sum(u, axis=0, keepdims=True),
         jnp.sum(u * u, axis=0, keepdims=True)], axis=0)

    @pl.when(pl.program_id(0) == 0)
    def _():
        st_ref[...] = blk

    @pl.when(pl.program_id(0) > 0)
    def _():
        st_ref[...] += blk


def _stats_tc(t, g34, E, D):
    row_spec = pl.BlockSpec((_BE, D), lambda i: (i, 0))
    return pl.pallas_call(
        _stats_body,
        grid=(E // _BE,),
        in_specs=[row_spec, row_spec],
        out_specs=[row_spec, pl.BlockSpec((2, D), lambda i: (0, 0))],
        out_shape=[
            jax.ShapeDtypeStruct((E, D), jnp.float32),
            jax.ShapeDtypeStruct((2, D), jnp.float32),
        ],
    )(t, g34)


def _node_body(nparts, lastlayer, *refs):
    i = 0
    x_ref = refs[i]; i += 1
    x1_ref = refs[i]; i += 1
    aggp_ref = refs[i]; i += 1
    counts_ref = refs[i]; i += 1
    gamma_ref = refs[i]; i += 1
    beta_ref = refs[i]; i += 1
    if not lastlayer:
        wv_ref = refs[i]; i += 1
        bv_ref = refs[i]; i += 1
    xout_ref = refs[i]; i += 1
    if not lastlayer:
        x1o_ref = refs[i]; i += 1
        x24o_ref = refs[i]; i += 1
        x3o_ref = refs[i]; i += 1

    agg = aggp_ref[0]
    for p in range(1, nparts):
        agg = agg + aggp_ref[p]
    agg = agg / counts_ref[...]
    pre = x1_ref[...] + agg
    mean = jnp.mean(pre, axis=0, keepdims=True)
    var = jnp.mean((pre - mean) ** 2, axis=0, keepdims=True)
    xn = gamma_ref[...] * (pre - mean) / jnp.sqrt(var + _EPS) + beta_ref[...]
    x = x_ref[...] + xn * jax.nn.sigmoid(xn)
    xout_ref[...] = x
    if not lastlayer:
        D = x.shape[1]
        mm = [
            jnp.dot(x, wv_ref[k], preferred_element_type=jnp.float32)
            + bv_ref[0, k][None, :]
            for k in range(4)
        ]
        x1o_ref[...] = mm[0]
        x24o_ref[:, :D] = mm[1]
        x24o_ref[:, D:] = mm[3]
        x3o_ref[...] = mm[2]


def _node_tc(x, x1, aggp, counts, gamma, beta, wv, bv, lastlayer, N, D):
    nparts = aggp.shape[0]
    args = [x, x1, aggp, counts, gamma.reshape(1, D), beta.reshape(1, D)]
    in_specs = [
        pl.BlockSpec((N, D), lambda: (0, 0)),
        pl.BlockSpec((N, D), lambda: (0, 0)),
        pl.BlockSpec((nparts, N, D), lambda: (0, 0, 0)),
        pl.BlockSpec((N, 1), lambda: (0, 0)),
        pl.BlockSpec((1, D), lambda: (0, 0)),
        pl.BlockSpec((1, D), lambda: (0, 0)),
    ]
    out_shape = [jax.ShapeDtypeStruct((N, D), jnp.float32)]
    out_specs = [pl.BlockSpec((N, D), lambda: (0, 0))]
    if not lastlayer:
        args += [wv, bv.reshape(1, 4, D)]
        in_specs += [
            pl.BlockSpec((4, D, D), lambda: (0, 0, 0)),
            pl.BlockSpec((1, 4, D), lambda: (0, 0, 0)),
        ]
        out_shape += [
            jax.ShapeDtypeStruct((N, D), jnp.float32),
            jax.ShapeDtypeStruct((N, 2 * D), jnp.float32),
            jax.ShapeDtypeStruct((N, D), jnp.float32),
        ]
        out_specs += [
            pl.BlockSpec((N, D), lambda: (0, 0)),
            pl.BlockSpec((N, 2 * D), lambda: (0, 0)),
            pl.BlockSpec((N, D), lambda: (0, 0)),
        ]
    return pl.pallas_call(
        functools.partial(_node_body, nparts, lastlayer),
        in_specs=in_specs,
        out_specs=out_specs,
        out_shape=out_shape,
    )(*args)


def _init_body(locs_ref, wi_ref, bi_ref, wv_ref, bv_ref,
               ne_ref, x1o_ref, x24o_ref, x3o_ref):
    ne = (
        jnp.dot(locs_ref[...], wi_ref[...], preferred_element_type=jnp.float32)
        + bi_ref[...]
    )
    ne_ref[...] = ne
    D = ne.shape[1]
    mm = [
        jnp.dot(ne, wv_ref[k], preferred_element_type=jnp.float32)
        + bv_ref[0, k][None, :]
        for k in range(4)
    ]
    x1o_ref[...] = mm[0]
    x24o_ref[:, :D] = mm[1]
    x24o_ref[:, D:] = mm[3]
    x3o_ref[...] = mm[2]


def _init_tc(locs, W_init, b_init, wv0, bv0, N, D):
    return pl.pallas_call(
        _init_body,
        in_specs=[
            pl.BlockSpec((N, 2), lambda: (0, 0)),
            pl.BlockSpec((2, D), lambda: (0, 0)),
            pl.BlockSpec((1, D), lambda: (0, 0)),
            pl.BlockSpec((4, D, D), lambda: (0, 0, 0)),
            pl.BlockSpec((1, 4, D), lambda: (0, 0, 0)),
        ],
        out_specs=[
            pl.BlockSpec((N, D), lambda: (0, 0)),
            pl.BlockSpec((N, D), lambda: (0, 0)),
            pl.BlockSpec((N, 2 * D), lambda: (0, 0)),
            pl.BlockSpec((N, D), lambda: (0, 0)),
        ],
        out_shape=[
            jax.ShapeDtypeStruct((N, D), jnp.float32),
            jax.ShapeDtypeStruct((N, D), jnp.float32),
            jax.ShapeDtypeStruct((N, 2 * D), jnp.float32),
            jax.ShapeDtypeStruct((N, D), jnp.float32),
        ],
    )(locs, W_init, b_init.reshape(1, D), wv0, bv0.reshape(1, 4, D))


def _sc_edge_body(src_hbm, dst_hbm, w_hbm, x24_hbm, x3_hbm,
                  g34_hbm, aggp_hbm,
                  srcv, dstv, wv, g24v, g3v, msgsv, zbuf, aggsh, *sems):
    D = _D
    s_ix = sems[0:_NB]
    s_w = sems[_NB:2 * _NB]
    s_g24 = sems[2 * _NB:3 * _NB]
    s_g3 = sems[3 * _NB:4 * _NB]
    s_sc = sems[4 * _NB:5 * _NB]
    s_st = sems[5 * _NB:6 * _NB]
    cid = lax.axis_index("c")
    sid = lax.axis_index("s")
    wid = sid * _NC + cid
    base_e = wid * _EPW
    zero = jnp.zeros((16,), jnp.float32)

    def zrow(j, carry):
        for c in range(D // 16):
            zbuf[j, pl.ds(c * 16, 16)] = zero
        return carry

    lax.fori_loop(0, 8, zrow, 0)
    row0 = sid * _RPS
    for i in range(_RPS // 8):
        pltpu.sync_copy(zbuf, aggsh.at[pl.ds(row0 + i * 8, 8)])

    @pl.when(sid == _NS - 1)
    def _zero_tail():
        pltpu.sync_copy(zbuf, aggsh.at[pl.ds(_NS * _RPS, 8)])
        pltpu.sync_copy(zbuf, aggsh.at[pl.ds(_NS * _RPS + 8, 8)])

    plsc.subcore_barrier()

    def _issue_idx(g, b):
        e0 = base_e + g * _K
        pltpu.async_copy(src_hbm.at[pl.ds(e0, _K)], srcv.at[b], s_ix[b])
        pltpu.async_copy(dst_hbm.at[pl.ds(e0, _K)], dstv.at[b], s_ix[b])

    def _wait_idx(b):
        pltpu.make_async_copy(src_hbm.at[pl.ds(0, _K)], srcv.at[b], s_ix[b]).wait()
        pltpu.make_async_copy(dst_hbm.at[pl.ds(0, _K)], dstv.at[b], s_ix[b]).wait()

    def _issue_gathers(g, b):
        e0 = base_e + g * _K
        pltpu.async_copy(w_hbm.at[pl.ds(e0, _K)], wv.at[b], s_w[b])
        pltpu.async_copy(x24_hbm.at[dstv.at[b]], g24v.at[b], s_g24[b])
        pltpu.async_copy(x3_hbm.at[srcv.at[b]], g3v.at[b], s_g3[b])

    def _wait_gathers(b):
        pltpu.make_async_copy(w_hbm.at[pl.ds(0, _K)], wv.at[b], s_w[b]).wait()
        pltpu.make_async_copy(
            x24_hbm.at[pl.ds(0, _K)], g24v.at[b], s_g24[b]).wait()
        pltpu.make_async_copy(
            x3_hbm.at[pl.ds(0, _K)], g3v.at[b], s_g3[b]).wait()

    def _wait_outs(b):
        pltpu.make_async_copy(
            msgsv.at[b], aggsh.at[pl.ds(0, _K)], s_sc[b]).wait()
        pltpu.make_async_copy(
            g3v.at[b], g34_hbm.at[pl.ds(0, _K)], s_st[b]).wait()

    def _compute(b):
        def row(j, rcarry):
            for c in range(D // 16):
                sl = pl.ds(c * 16, 16)
                wvec = wv[b, j, sl]
                sig = 1.0 / (1.0 + jnp.exp(-wvec))
                msgsv[b, j, sl] = sig * g24v[b, j, sl]
                g3v[b, j, sl] = g3v[b, j, sl] + g24v[b, j, pl.ds(D + c * 16, 16)]
            return rcarry

        lax.fori_loop(0, _K, row, 0, unroll=2)

    def _issue_outs(g, b):
        e0 = base_e + g * _K
        pltpu.async_copy(msgsv.at[b], aggsh.at[srcv.at[b]], s_sc[b], add=True)
        pltpu.async_copy(g3v.at[b], g34_hbm.at[pl.ds(e0, _K)], s_st[b])

    _issue_idx(0, 0)
    _issue_idx(1, 1)
    _issue_idx(2, 2)
    _wait_idx(0)
    _issue_gathers(0, 0)
    _wait_idx(1)
    _issue_gathers(1, 1)

    def _slot(g, s):
        b = s & (_NB - 1)
        b2 = (s + 2) & (_NB - 1)
        b3 = (s + 3) & (_NB - 1)

        @pl.when(g < _NCHUNK)
        def _():
            _wait_gathers(b)
            _compute(b)
            _issue_outs(g, b)

        @pl.when((g >= 1) & (g <= _NCHUNK))
        def _():
            _wait_outs(b3)

        @pl.when(g + 3 < _NCHUNK)
        def _():
            _issue_idx(g + 3, b3)

        @pl.when(g + 2 < _NCHUNK)
        def _():
            _wait_idx(b2)
            _issue_gathers(g + 2, b2)

    def body4(i, carry):
        for s in range(_NB):
            _slot(i * _NB + s, s)
        return carry

    lax.fori_loop(0, (_NCHUNK + _NB + 1) // _NB, body4, 0)

    plsc.subcore_barrier()
    for i in range(_RPS // 104):
        r = row0 + i * 104
        pltpu.sync_copy(aggsh.at[pl.ds(r, 104)], aggp_hbm.at[cid, pl.ds(r, 104)])

    @pl.when(sid == _NS - 1)
    def _flush_tail():
        r = _NS * _RPS
        pltpu.sync_copy(aggsh.at[pl.ds(r, 16)], aggp_hbm.at[cid, pl.ds(r, 16)])


_sc_edge_fn = None


def _sc_edge(*args):
    global _sc_edge_fn
    if _sc_edge_fn is None:
        _sc_edge_fn = _make_sc_edge()
    return _sc_edge_fn(*args)


def _make_sc_edge():
    return functools.partial(
        pl.kernel,
        out_type=[
            jax.ShapeDtypeStruct((_E, _D), jnp.float32),
            jax.ShapeDtypeStruct((_NC, _N, _D), jnp.float32),
        ],
        mesh=plsc.VectorSubcoreMesh(
            core_axis_name="c", subcore_axis_name="s", num_cores=_NC),
        scratch_types=[
            pltpu.VMEM((_NB, _K), jnp.int32),
            pltpu.VMEM((_NB, _K), jnp.int32),
            pltpu.VMEM((_NB, _K, _D), jnp.float32),
            pltpu.VMEM((_NB, _K, 2 * _D), jnp.float32),
            pltpu.VMEM((_NB, _K, _D), jnp.float32),
            pltpu.VMEM((_NB, _K, _D), jnp.float32),
            pltpu.VMEM((8, _D), jnp.float32),
            pltpu.VMEM_SHARED((_N, _D), jnp.float32),
        ] + [pltpu.SemaphoreType.DMA] * (6 * _NB),
    )(_sc_edge_body)


def _bn_affine(s, ss, count, gamma, beta):
    mean = s / count
    var = ss / count - mean * mean
    inv = gamma / jnp.sqrt(var + _EPS)
    return inv, beta - mean * inv


def kernel(locs, edge_index, W_init, b_init, W_edge, b_edge, Wv, bv, We, be,
           gamma_v, beta_v, gamma_e, beta_e):
    N, D = locs.shape[0], W_init.shape[1]
    E = edge_index.shape[1]
    L = Wv.shape[0]
    src = edge_index[0]
    dst = edge_index[1]

    dl = locs[src] - locs[dst]
    dist = jnp.sqrt(dl[:, 0] ** 2 + dl[:, 1] ** 2 + 1e-12)
    ones = jnp.ones((E,), jnp.float32)
    counts = jnp.maximum(
        jax.ops.segment_sum(ones, src, num_segments=N), 1.0
    ).reshape(N, 1)

    dist2 = dist.reshape(E, 1)
    wedge = W_edge.reshape(1, D)
    bedge = b_edge.reshape(1, D)

    node_embed, x1, x24, x3 = _init_tc(locs, W_init, b_init, Wv[0], bv[0], N, D)
    x = node_embed

    w_prev = None
    u_prev = None
    su = ssu = None
    for l in range(L):
        if l == 0:
            (w_cur,) = _edge_tc(
                True, False, True, dist2, wedge, bedge, E=E, D=D)
        elif l == 1:
            A, B = _bn_affine(su, ssu, float(E), gamma_e[l - 1], beta_e[l - 1])
            (w_cur,) = _edge_tc(
                True, True, True,
                dist2, wedge, bedge, u_prev, A.reshape(1, D), B.reshape(1, D),
                E=E, D=D)
        else:
            A, B = _bn_affine(su, ssu, float(E), gamma_e[l - 1], beta_e[l - 1])
            (w_cur,) = _edge_tc(
                False, True, True,
                w_prev, u_prev, A.reshape(1, D), B.reshape(1, D), E=E, D=D)

        g34, aggp = _sc_edge(src, dst, w_cur, x24, x3)
        t = _matmul_tc(w_cur, We[l], be[l].reshape(1, D), E, D)
        u, st = _stats_tc(t, g34, E, D)
        su, ssu = st[0], st[1]

        lastlayer = l == L - 1
        if lastlayer:
            x = _node_tc(x, x1, aggp, counts, gamma_v[l], beta_v[l],
                         None, None, True, N, D)[0]
        else:
            x, x1, x24, x3 = _node_tc(
                x, x1, aggp, counts, gamma_v[l], beta_v[l],
                Wv[l + 1], bv[l + 1], False, N, D)
        w_prev = w_cur
        u_prev = u

    A, B = _bn_affine(su, ssu, float(E), gamma_e[L - 1], beta_e[L - 1])
    (w_final,) = _edge_tc(
        False, True, True,
        w_prev, u_prev, A.reshape(1, D), B.reshape(1, D), E=E, D=D)
    return (x, w_final, node_embed)


# speedup vs baseline: 2.3357x; 1.7701x over previous
import functools

import jax
import jax.numpy as jnp
from jax import lax
from jax.experimental import pallas as pl
from jax.experimental.pallas import tpu as pltpu
from jax.experimental.pallas import tpu_sc as plsc

_BE = 3200
_EPS = 1e-5

_N = 10000
_E = 320000
_D = 128
_NC = 1
_NS = 16
_NW = _NC * _NS
_EPW = _E // _NW
_K = 16
_NCHUNK = _EPW // _K
_NB = 4
_RPS = 624


def _edge_tc_body(first, has_u, sig_out, *refs):
    i = 0
    if first:
        dist_ref = refs[i]; i += 1
        wedge_ref = refs[i]; i += 1
        bedge_ref = refs[i]; i += 1
    else:
        wprev_ref = refs[i]; i += 1
    if has_u:
        u_ref = refs[i]; i += 1
        a_ref = refs[i]; i += 1
        b_ref = refs[i]; i += 1
    wout_ref = refs[i]; i += 1
    if sig_out:
        sout_ref = refs[i]; i += 1

    if first:
        w_prev = dist_ref[...] * wedge_ref[...] + bedge_ref[...]
    else:
        w_prev = wprev_ref[...]
    if has_u:
        z = u_ref[...] * a_ref[...] + b_ref[...]
        w = w_prev + z * jax.nn.sigmoid(z)
    else:
        w = w_prev
    wout_ref[...] = w
    if sig_out:
        sout_ref[...] = jax.nn.sigmoid(w)


def _edge_tc(first, has_u, sig_out, *args, E, D):
    grid = (E // _BE,)
    row_spec = pl.BlockSpec((_BE, D), lambda i: (i, 0))
    dist_spec = pl.BlockSpec((_BE, 1), lambda i: (i, 0))
    vec_spec = pl.BlockSpec((1, D), lambda i: (0, 0))
    in_specs = []
    if first:
        in_specs += [dist_spec, vec_spec, vec_spec]
    else:
        in_specs += [row_spec]
    if has_u:
        in_specs += [row_spec, vec_spec, vec_spec]
    n_out = 2 if sig_out else 1
    out_specs = [row_spec] * n_out
    out_shape = [jax.ShapeDtypeStruct((E, D), jnp.float32)] * n_out
    return pl.pallas_call(
        functools.partial(_edge_tc_body, first, has_u, sig_out),
        grid=grid,
        in_specs=in_specs,
        out_specs=out_specs,
        out_shape=out_shape,
    )(*args)


def _matmul_body(w_ref, we_ref, be_ref, t_ref):
    t_ref[...] = (
        jnp.dot(w_ref[...], we_ref[...], preferred_element_type=jnp.float32)
        + be_ref[...]
    )


def _matmul_tc(w, We, be, E, D):
    row_spec = pl.BlockSpec((_BE, D), lambda i: (i, 0))
    return pl.pallas_call(
        _matmul_body,
        grid=(E // _BE,),
        in_specs=[
            row_spec,
            pl.BlockSpec((D, D), lambda i: (0, 0)),
            pl.BlockSpec((1, D), lambda i: (0, 0)),
        ],
        out_specs=row_spec,
        out_shape=jax.ShapeDtypeStruct((E, D), jnp.float32),
    )(w, We, be)


def _stats_body(t_ref, g_ref, u_ref, st_ref):
    u = t_ref[...] + g_ref[...]
    u_ref[...] = u
    blk = jnp.concatenate(
        [jnp.sum(u, axis=0, keepdims=True),
         jnp.sum(u * u, axis=0, keepdims=True)], axis=0)

    @pl.when(pl.program_id(0) == 0)
    def _():
        st_ref[...] = blk

    @pl.when(pl.program_id(0) > 0)
    def _():
        st_ref[...] += blk


def _stats_tc(t, g34, E, D):
    row_spec = pl.BlockSpec((_BE, D), lambda i: (i, 0))
    return pl.pallas_call(
        _stats_body,
        grid=(E // _BE,),
        in_specs=[row_spec, row_spec],
        out_specs=[row_spec, pl.BlockSpec((2, D), lambda i: (0, 0))],
        out_shape=[
            jax.ShapeDtypeStruct((E, D), jnp.float32),
            jax.ShapeDtypeStruct((2, D), jnp.float32),
        ],
    )(t, g34)


def _node_body(nparts, lastlayer, *refs):
    i = 0
    x_ref = refs[i]; i += 1
    x1_ref = refs[i]; i += 1
    aggp_ref = refs[i]; i += 1
    counts_ref = refs[i]; i += 1
    gamma_ref = refs[i]; i += 1
    beta_ref = refs[i]; i += 1
    if not lastlayer:
        wv_ref = refs[i]; i += 1
        bv_ref = refs[i]; i += 1
    xout_ref = refs[i]; i += 1
    if not lastlayer:
        x1o_ref = refs[i]; i += 1
        x24o_ref = refs[i]; i += 1
        x3o_ref = refs[i]; i += 1

    agg = aggp_ref[0]
    for p in range(1, nparts):
        agg = agg + aggp_ref[p]
    agg = agg / counts_ref[...]
    pre = x1_ref[...] + agg
    mean = jnp.mean(pre, axis=0, keepdims=True)
    var = jnp.mean((pre - mean) ** 2, axis=0, keepdims=True)
    xn = gamma_ref[...] * (pre - mean) / jnp.sqrt(var + _EPS) + beta_ref[...]
    x = x_ref[...] + xn * jax.nn.sigmoid(xn)
    xout_ref[...] = x
    if not lastlayer:
        D = x.shape[1]
        mm = [
            jnp.dot(x, wv_ref[k], preferred_element_type=jnp.float32)
            + bv_ref[0, k][None, :]
            for k in range(4)
        ]
        x1o_ref[...] = mm[0]
        x24o_ref[:, :D] = mm[1]
        x24o_ref[:, D:] = mm[3]
        x3o_ref[...] = mm[2]


def _node_tc(x, x1, aggp, counts, gamma, beta, wv, bv, lastlayer, N, D):
    nparts = aggp.shape[0]
    args = [x, x1, aggp, counts, gamma.reshape(1, D), beta.reshape(1, D)]
    in_specs = [
        pl.BlockSpec((N, D), lambda: (0, 0)),
        pl.BlockSpec((N, D), lambda: (0, 0)),
        pl.BlockSpec((nparts, N, D), lambda: (0, 0, 0)),
        pl.BlockSpec((N, 1), lambda: (0, 0)),
        pl.BlockSpec((1, D), lambda: (0, 0)),
        pl.BlockSpec((1, D), lambda: (0, 0)),
    ]
    out_shape = [jax.ShapeDtypeStruct((N, D), jnp.float32)]
    out_specs = [pl.BlockSpec((N, D), lambda: (0, 0))]
    if not lastlayer:
        args += [wv, bv.reshape(1, 4, D)]
        in_specs += [
            pl.BlockSpec((4, D, D), lambda: (0, 0, 0)),
            pl.BlockSpec((1, 4, D), lambda: (0, 0, 0)),
        ]
        out_shape += [
            jax.ShapeDtypeStruct((N, D), jnp.float32),
            jax.ShapeDtypeStruct((N, 2 * D), jnp.float32),
            jax.ShapeDtypeStruct((N, D), jnp.float32),
        ]
        out_specs += [
            pl.BlockSpec((N, D), lambda: (0, 0)),
            pl.BlockSpec((N, 2 * D), lambda: (0, 0)),
            pl.BlockSpec((N, D), lambda: (0, 0)),
        ]
    return pl.pallas_call(
        functools.partial(_node_body, nparts, lastlayer),
        in_specs=in_specs,
        out_specs=out_specs,
        out_shape=out_shape,
    )(*args)


def _init_body(locs_ref, wi_ref, bi_ref, wv_ref, bv_ref,
               ne_ref, x1o_ref, x24o_ref, x3o_ref):
    ne = (
        jnp.dot(locs_ref[...], wi_ref[...], preferred_element_type=jnp.float32)
        + bi_ref[...]
    )
    ne_ref[...] = ne
    D = ne.shape[1]
    mm = [
        jnp.dot(ne, wv_ref[k], preferred_element_type=jnp.float32)
        + bv_ref[0, k][None, :]
        for k in range(4)
    ]
    x1o_ref[...] = mm[0]
    x24o_ref[:, :D] = mm[1]
    x24o_ref[:, D:] = mm[3]
    x3o_ref[...] = mm[2]


def _init_tc(locs, W_init, b_init, wv0, bv0, N, D):
    return pl.pallas_call(
        _init_body,
        in_specs=[
            pl.BlockSpec((N, 2), lambda: (0, 0)),
            pl.BlockSpec((2, D), lambda: (0, 0)),
            pl.BlockSpec((1, D), lambda: (0, 0)),
            pl.BlockSpec((4, D, D), lambda: (0, 0, 0)),
            pl.BlockSpec((1, 4, D), lambda: (0, 0, 0)),
        ],
        out_specs=[
            pl.BlockSpec((N, D), lambda: (0, 0)),
            pl.BlockSpec((N, D), lambda: (0, 0)),
            pl.BlockSpec((N, 2 * D), lambda: (0, 0)),
            pl.BlockSpec((N, D), lambda: (0, 0)),
        ],
        out_shape=[
            jax.ShapeDtypeStruct((N, D), jnp.float32),
            jax.ShapeDtypeStruct((N, D), jnp.float32),
            jax.ShapeDtypeStruct((N, 2 * D), jnp.float32),
            jax.ShapeDtypeStruct((N, D), jnp.float32),
        ],
    )(locs, W_init, b_init.reshape(1, D), wv0, bv0.reshape(1, 4, D))


def _sc_edge_body(src_hbm, dst_hbm, sig_hbm, x24_hbm, x3_hbm,
                  g34_hbm, aggp_hbm,
                  srcv, dstv, sigv, g24v, g3v, zbuf, aggsh, *sems):
    D = _D
    s_ix = sems[0:_NB]
    s_w = sems[_NB:2 * _NB]
    s_g24 = sems[2 * _NB:3 * _NB]
    s_g3 = sems[3 * _NB:4 * _NB]
    s_sc = sems[4 * _NB:5 * _NB]
    s_st = sems[5 * _NB:6 * _NB]
    cid = lax.axis_index("c")
    sid = lax.axis_index("s")
    wid = sid * _NC + cid
    base_e = wid * _EPW
    zero = jnp.zeros((16,), jnp.float32)

    def zrow(j, carry):
        for c in range(D // 16):
            zbuf[j, pl.ds(c * 16, 16)] = zero
        return carry

    lax.fori_loop(0, 8, zrow, 0)
    row0 = sid * _RPS
    for i in range(_RPS // 8):
        pltpu.sync_copy(zbuf, aggsh.at[pl.ds(row0 + i * 8, 8)])

    @pl.when(sid == _NS - 1)
    def _zero_tail():
        pltpu.sync_copy(zbuf, aggsh.at[pl.ds(_NS * _RPS, 8)])
        pltpu.sync_copy(zbuf, aggsh.at[pl.ds(_NS * _RPS + 8, 8)])

    plsc.subcore_barrier()

    def _issue_idx(g, b):
        e0 = base_e + g * _K
        pltpu.async_copy(src_hbm.at[pl.ds(e0, _K)], srcv.at[b], s_ix[b])
        pltpu.async_copy(dst_hbm.at[pl.ds(e0, _K)], dstv.at[b], s_ix[b])

    def _wait_idx(b):
        pltpu.make_async_copy(src_hbm.at[pl.ds(0, _K)], srcv.at[b], s_ix[b]).wait()
        pltpu.make_async_copy(dst_hbm.at[pl.ds(0, _K)], dstv.at[b], s_ix[b]).wait()

    def _issue_gathers(g, b):
        e0 = base_e + g * _K
        pltpu.async_copy(sig_hbm.at[pl.ds(e0, _K)], sigv.at[b], s_w[b])
        pltpu.async_copy(x24_hbm.at[dstv.at[b]], g24v.at[b], s_g24[b])
        pltpu.async_copy(x3_hbm.at[srcv.at[b]], g3v.at[b], s_g3[b])

    def _wait_gathers(b):
        pltpu.make_async_copy(sig_hbm.at[pl.ds(0, _K)], sigv.at[b], s_w[b]).wait()
        pltpu.make_async_copy(
            x24_hbm.at[pl.ds(0, _K)], g24v.at[b], s_g24[b]).wait()
        pltpu.make_async_copy(
            x3_hbm.at[pl.ds(0, _K)], g3v.at[b], s_g3[b]).wait()

    def _wait_outs(b):
        pltpu.make_async_copy(
            sigv.at[b], aggsh.at[pl.ds(0, _K)], s_sc[b]).wait()
        pltpu.make_async_copy(
            g3v.at[b], g34_hbm.at[pl.ds(0, _K)], s_st[b]).wait()

    def _compute(b):
        def row(j, rcarry):
            for c in range(D // 16):
                sl = pl.ds(c * 16, 16)
                sigv[b, j, sl] = sigv[b, j, sl] * g24v[b, j, sl]
                g3v[b, j, sl] = g3v[b, j, sl] + g24v[b, j, pl.ds(D + c * 16, 16)]
            return rcarry

        lax.fori_loop(0, _K, row, 0, unroll=2)

    def _issue_outs(g, b):
        e0 = base_e + g * _K
        pltpu.async_copy(sigv.at[b], aggsh.at[srcv.at[b]], s_sc[b], add=True)
        pltpu.async_copy(g3v.at[b], g34_hbm.at[pl.ds(e0, _K)], s_st[b])

    _issue_idx(0, 0)
    _issue_idx(1, 1)
    _issue_idx(2, 2)
    _wait_idx(0)
    _issue_gathers(0, 0)
    _wait_idx(1)
    _issue_gathers(1, 1)

    def _slot(g, s):
        b = s & (_NB - 1)
        b2 = (s + 2) & (_NB - 1)
        b3 = (s + 3) & (_NB - 1)

        @pl.when(g < _NCHUNK)
        def _():
            _wait_gathers(b)
            _compute(b)
            _issue_outs(g, b)

        @pl.when((g >= 1) & (g <= _NCHUNK))
        def _():
            _wait_outs(b3)

        @pl.when(g + 3 < _NCHUNK)
        def _():
            _issue_idx(g + 3, b3)

        @pl.when(g + 2 < _NCHUNK)
        def _():
            _wait_idx(b2)
            _issue_gathers(g + 2, b2)

    def body4(i, carry):
        for s in range(_NB):
            _slot(i * _NB + s, s)
        return carry

    lax.fori_loop(0, (_NCHUNK + _NB + 1) // _NB, body4, 0)

    plsc.subcore_barrier()
    for i in range(_RPS // 104):
        r = row0 + i * 104
        pltpu.sync_copy(aggsh.at[pl.ds(r, 104)], aggp_hbm.at[cid, pl.ds(r, 104)])

    @pl.when(sid == _NS - 1)
    def _flush_tail():
        r = _NS * _RPS
        pltpu.sync_copy(aggsh.at[pl.ds(r, 16)], aggp_hbm.at[cid, pl.ds(r, 16)])


_sc_edge_fn = None


def _sc_edge(*args):
    global _sc_edge_fn
    if _sc_edge_fn is None:
        _sc_edge_fn = _make_sc_edge()
    return _sc_edge_fn(*args)


def _make_sc_edge():
    return functools.partial(
        pl.kernel,
        out_type=[
            jax.ShapeDtypeStruct((_E, _D), jnp.float32),
            jax.ShapeDtypeStruct((_NC, _N, _D), jnp.float32),
        ],
        mesh=plsc.VectorSubcoreMesh(
            core_axis_name="c", subcore_axis_name="s", num_cores=_NC),
        scratch_types=[
            pltpu.VMEM((_NB, _K), jnp.int32),
            pltpu.VMEM((_NB, _K), jnp.int32),
            pltpu.VMEM((_NB, _K, _D), jnp.float32),
            pltpu.VMEM((_NB, _K, 2 * _D), jnp.float32),
            pltpu.VMEM((_NB, _K, _D), jnp.float32),
            pltpu.VMEM((8, _D), jnp.float32),
            pltpu.VMEM_SHARED((_N, _D), jnp.float32),
        ] + [pltpu.SemaphoreType.DMA] * (6 * _NB),
    )(_sc_edge_body)


def _bn_affine(s, ss, count, gamma, beta):
    mean = s / count
    var = ss / count - mean * mean
    inv = gamma / jnp.sqrt(var + _EPS)
    return inv, beta - mean * inv


def kernel(locs, edge_index, W_init, b_init, W_edge, b_edge, Wv, bv, We, be,
           gamma_v, beta_v, gamma_e, beta_e):
    N, D = locs.shape[0], W_init.shape[1]
    E = edge_index.shape[1]
    L = Wv.shape[0]
    src = edge_index[0]
    dst = edge_index[1]

    dl = locs[src] - locs[dst]
    dist = jnp.sqrt(dl[:, 0] ** 2 + dl[:, 1] ** 2 + 1e-12)
    ones = jnp.ones((E,), jnp.float32)
    counts = jnp.maximum(
        jax.ops.segment_sum(ones, src, num_segments=N), 1.0
    ).reshape(N, 1)

    dist2 = dist.reshape(E, 1)
    wedge = W_edge.reshape(1, D)
    bedge = b_edge.reshape(1, D)

    node_embed, x1, x24, x3 = _init_tc(locs, W_init, b_init, Wv[0], bv[0], N, D)
    x = node_embed

    w_prev = None
    u_prev = None
    su = ssu = None
    for l in range(L):
        if l == 0:
            w_cur, sig = _edge_tc(
                True, False, True, dist2, wedge, bedge, E=E, D=D)
        elif l == 1:
            A, B = _bn_affine(su, ssu, float(E), gamma_e[l - 1], beta_e[l - 1])
            w_cur, sig = _edge_tc(
                True, True, True,
                dist2, wedge, bedge, u_prev, A.reshape(1, D), B.reshape(1, D),
                E=E, D=D)
        else:
            A, B = _bn_affine(su, ssu, float(E), gamma_e[l - 1], beta_e[l - 1])
            w_cur, sig = _edge_tc(
                False, True, True,
                w_prev, u_prev, A.reshape(1, D), B.reshape(1, D), E=E, D=D)

        g34, aggp = _sc_edge(src, dst, sig, x24, x3)
        t = _matmul_tc(w_cur, We[l], be[l].reshape(1, D), E, D)
        u, st = _stats_tc(t, g34, E, D)
        su, ssu = st[0], st[1]

        lastlayer = l == L - 1
        if lastlayer:
            x = _node_tc(x, x1, aggp, counts, gamma_v[l], beta_v[l],
                         None, None, True, N, D)[0]
        else:
            x, x1, x24, x3 = _node_tc(
                x, x1, aggp, counts, gamma_v[l], beta_v[l],
                Wv[l + 1], bv[l + 1], False, N, D)
        w_prev = w_cur
        u_prev = u

    A, B = _bn_affine(su, ssu, float(E), gamma_e[L - 1], beta_e[L - 1])
    (w_final,) = _edge_tc(
        False, True, False,
        w_prev, u_prev, A.reshape(1, D), B.reshape(1, D), E=E, D=D)
    return (x, w_final, node_embed)


# speedup vs baseline: 2.9867x; 1.2787x over previous
import functools

import jax
import jax.numpy as jnp
from jax import lax
from jax.experimental import pallas as pl
from jax.experimental.pallas import tpu as pltpu
from jax.experimental.pallas import tpu_sc as plsc

_BE = 3200
_EPS = 1e-5

_N = 10000
_E = 320000
_D = 128
_NC = 1
_NS = 16
_NW = _NC * _NS
_EPW = _E // _NW
_K = 16
_NCHUNK = _EPW // _K
_NB = 4
_RPS = 624


def _edge_tc_body(first, has_u, sig_out, *refs):
    i = 0
    if first:
        dist_ref = refs[i]; i += 1
        wedge_ref = refs[i]; i += 1
        bedge_ref = refs[i]; i += 1
    else:
        wprev_ref = refs[i]; i += 1
    if has_u:
        u_ref = refs[i]; i += 1
        a_ref = refs[i]; i += 1
        b_ref = refs[i]; i += 1
    wout_ref = refs[i]; i += 1
    if sig_out:
        sout_ref = refs[i]; i += 1

    if first:
        w_prev = jnp.sqrt(dist_ref[...]) * wedge_ref[...] + bedge_ref[...]
    else:
        w_prev = wprev_ref[...]
    if has_u:
        z = u_ref[...] * a_ref[...] + b_ref[...]
        w = w_prev + z * jax.nn.sigmoid(z)
    else:
        w = w_prev
    wout_ref[...] = w
    if sig_out:
        sout_ref[...] = jax.nn.sigmoid(w)


def _edge_tc(first, has_u, sig_out, *args, E, D):
    grid = (E // _BE,)
    row_spec = pl.BlockSpec((_BE, D), lambda i: (i, 0))
    dist_spec = pl.BlockSpec((_BE, 1), lambda i: (i, 0))
    vec_spec = pl.BlockSpec((1, D), lambda i: (0, 0))
    in_specs = []
    if first:
        in_specs += [dist_spec, vec_spec, vec_spec]
    else:
        in_specs += [row_spec]
    if has_u:
        in_specs += [row_spec, vec_spec, vec_spec]
    n_out = 2 if sig_out else 1
    out_specs = [row_spec] * n_out
    out_shape = [jax.ShapeDtypeStruct((E, D), jnp.float32)] * n_out
    return pl.pallas_call(
        functools.partial(_edge_tc_body, first, has_u, sig_out),
        grid=grid,
        in_specs=in_specs,
        out_specs=out_specs,
        out_shape=out_shape,
    )(*args)


def _matmul_body(w_ref, we_ref, be_ref, t_ref):
    t_ref[...] = (
        jnp.dot(w_ref[...], we_ref[...], preferred_element_type=jnp.float32)
        + be_ref[...]
    )


def _matmul_tc(w, We, be, E, D):
    row_spec = pl.BlockSpec((_BE, D), lambda i: (i, 0))
    return pl.pallas_call(
        _matmul_body,
        grid=(E // _BE,),
        in_specs=[
            row_spec,
            pl.BlockSpec((D, D), lambda i: (0, 0)),
            pl.BlockSpec((1, D), lambda i: (0, 0)),
        ],
        out_specs=row_spec,
        out_shape=jax.ShapeDtypeStruct((E, D), jnp.float32),
    )(w, We, be)


def _stats_body(t_ref, g_ref, u_ref, st_ref):
    u = t_ref[...] + g_ref[...]
    u_ref[...] = u
    blk = jnp.concatenate(
        [jnp.sum(u, axis=0, keepdims=True),
         jnp.sum(u * u, axis=0, keepdims=True)], axis=0)

    @pl.when(pl.program_id(0) == 0)
    def _():
        st_ref[...] = blk

    @pl.when(pl.program_id(0) > 0)
    def _():
        st_ref[...] += blk


def _stats_tc(t, g34, E, D):
    row_spec = pl.BlockSpec((_BE, D), lambda i: (i, 0))
    return pl.pallas_call(
        _stats_body,
        grid=(E // _BE,),
        in_specs=[row_spec, row_spec],
        out_specs=[row_spec, pl.BlockSpec((2, D), lambda i: (0, 0))],
        out_shape=[
            jax.ShapeDtypeStruct((E, D), jnp.float32),
            jax.ShapeDtypeStruct((2, D), jnp.float32),
        ],
    )(t, g34)


def _node_body(nparts, lastlayer, *refs):
    i = 0
    x_ref = refs[i]; i += 1
    x1_ref = refs[i]; i += 1
    aggp_ref = refs[i]; i += 1
    counts_ref = refs[i]; i += 1
    gamma_ref = refs[i]; i += 1
    beta_ref = refs[i]; i += 1
    if not lastlayer:
        wv_ref = refs[i]; i += 1
        bv_ref = refs[i]; i += 1
    xout_ref = refs[i]; i += 1
    if not lastlayer:
        x1o_ref = refs[i]; i += 1
        x24o_ref = refs[i]; i += 1
        x3o_ref = refs[i]; i += 1

    agg = aggp_ref[0]
    for p in range(1, nparts):
        agg = agg + aggp_ref[p]
    agg = agg / counts_ref[...]
    pre = x1_ref[...] + agg
    mean = jnp.mean(pre, axis=0, keepdims=True)
    var = jnp.mean((pre - mean) ** 2, axis=0, keepdims=True)
    xn = gamma_ref[...] * (pre - mean) / jnp.sqrt(var + _EPS) + beta_ref[...]
    x = x_ref[...] + xn * jax.nn.sigmoid(xn)
    xout_ref[...] = x
    if not lastlayer:
        D = x.shape[1]
        mm = [
            jnp.dot(x, wv_ref[k], preferred_element_type=jnp.float32)
            + bv_ref[0, k][None, :]
            for k in range(4)
        ]
        x1o_ref[...] = mm[0]
        x24o_ref[:, :D] = mm[1]
        x24o_ref[:, D:] = mm[3]
        x3o_ref[...] = mm[2]


def _node_tc(x, x1, aggp, counts, gamma, beta, wv, bv, lastlayer, N, D):
    nparts = aggp.shape[0]
    args = [x, x1, aggp, counts, gamma.reshape(1, D), beta.reshape(1, D)]
    in_specs = [
        pl.BlockSpec((N, D), lambda: (0, 0)),
        pl.BlockSpec((N, D), lambda: (0, 0)),
        pl.BlockSpec((nparts, N, D), lambda: (0, 0, 0)),
        pl.BlockSpec((N, 1), lambda: (0, 0)),
        pl.BlockSpec((1, D), lambda: (0, 0)),
        pl.BlockSpec((1, D), lambda: (0, 0)),
    ]
    out_shape = [jax.ShapeDtypeStruct((N, D), jnp.float32)]
    out_specs = [pl.BlockSpec((N, D), lambda: (0, 0))]
    if not lastlayer:
        args += [wv, bv.reshape(1, 4, D)]
        in_specs += [
            pl.BlockSpec((4, D, D), lambda: (0, 0, 0)),
            pl.BlockSpec((1, 4, D), lambda: (0, 0, 0)),
        ]
        out_shape += [
            jax.ShapeDtypeStruct((N, D), jnp.float32),
            jax.ShapeDtypeStruct((N, 2 * D), jnp.float32),
            jax.ShapeDtypeStruct((N, D), jnp.float32),
        ]
        out_specs += [
            pl.BlockSpec((N, D), lambda: (0, 0)),
            pl.BlockSpec((N, 2 * D), lambda: (0, 0)),
            pl.BlockSpec((N, D), lambda: (0, 0)),
        ]
    return pl.pallas_call(
        functools.partial(_node_body, nparts, lastlayer),
        in_specs=in_specs,
        out_specs=out_specs,
        out_shape=out_shape,
    )(*args)


def _init_body(locs_ref, wi_ref, bi_ref, wv_ref, bv_ref,
               ne_ref, x1o_ref, x24o_ref, x3o_ref):
    ne = (
        jnp.dot(locs_ref[...], wi_ref[...], preferred_element_type=jnp.float32)
        + bi_ref[...]
    )
    ne_ref[...] = ne
    D = ne.shape[1]
    mm = [
        jnp.dot(ne, wv_ref[k], preferred_element_type=jnp.float32)
        + bv_ref[0, k][None, :]
        for k in range(4)
    ]
    x1o_ref[...] = mm[0]
    x24o_ref[:, :D] = mm[1]
    x24o_ref[:, D:] = mm[3]
    x3o_ref[...] = mm[2]


def _init_tc(locs, W_init, b_init, wv0, bv0, N, D):
    return pl.pallas_call(
        _init_body,
        in_specs=[
            pl.BlockSpec((N, 2), lambda: (0, 0)),
            pl.BlockSpec((2, D), lambda: (0, 0)),
            pl.BlockSpec((1, D), lambda: (0, 0)),
            pl.BlockSpec((4, D, D), lambda: (0, 0, 0)),
            pl.BlockSpec((1, 4, D), lambda: (0, 0, 0)),
        ],
        out_specs=[
            pl.BlockSpec((N, D), lambda: (0, 0)),
            pl.BlockSpec((N, D), lambda: (0, 0)),
            pl.BlockSpec((N, 2 * D), lambda: (0, 0)),
            pl.BlockSpec((N, D), lambda: (0, 0)),
        ],
        out_shape=[
            jax.ShapeDtypeStruct((N, D), jnp.float32),
            jax.ShapeDtypeStruct((N, D), jnp.float32),
            jax.ShapeDtypeStruct((N, 2 * D), jnp.float32),
            jax.ShapeDtypeStruct((N, D), jnp.float32),
        ],
    )(locs, W_init, b_init.reshape(1, D), wv0, bv0.reshape(1, 4, D))


def _sc_edge_body(src_hbm, dst_hbm, sig_hbm, x24_hbm, x3_hbm,
                  g34_hbm, aggp_hbm,
                  srcv, dstv, sigv, g24v, g3v, zbuf, aggsh, *sems):
    D = _D
    s_ix = sems[0:_NB]
    s_w = sems[_NB:2 * _NB]
    s_g24 = sems[2 * _NB:3 * _NB]
    s_g3 = sems[3 * _NB:4 * _NB]
    s_sc = sems[4 * _NB:5 * _NB]
    s_st = sems[5 * _NB:6 * _NB]
    cid = lax.axis_index("c")
    sid = lax.axis_index("s")
    wid = sid * _NC + cid
    base_e = wid * _EPW
    zero = jnp.zeros((16,), jnp.float32)

    def zrow(j, carry):
        for c in range(D // 16):
            zbuf[j, pl.ds(c * 16, 16)] = zero
        return carry

    lax.fori_loop(0, 8, zrow, 0)
    row0 = sid * _RPS
    for i in range(_RPS // 8):
        pltpu.sync_copy(zbuf, aggsh.at[pl.ds(row0 + i * 8, 8)])

    @pl.when(sid == _NS - 1)
    def _zero_tail():
        pltpu.sync_copy(zbuf, aggsh.at[pl.ds(_NS * _RPS, 8)])
        pltpu.sync_copy(zbuf, aggsh.at[pl.ds(_NS * _RPS + 8, 8)])

    plsc.subcore_barrier()

    def _issue_idx(g, b):
        e0 = base_e + g * _K
        pltpu.async_copy(src_hbm.at[pl.ds(e0, _K)], srcv.at[b], s_ix[b])
        pltpu.async_copy(dst_hbm.at[pl.ds(e0, _K)], dstv.at[b], s_ix[b])

    def _wait_idx(b):
        pltpu.make_async_copy(src_hbm.at[pl.ds(0, _K)], srcv.at[b], s_ix[b]).wait()
        pltpu.make_async_copy(dst_hbm.at[pl.ds(0, _K)], dstv.at[b], s_ix[b]).wait()

    def _issue_gathers(g, b):
        e0 = base_e + g * _K
        pltpu.async_copy(sig_hbm.at[pl.ds(e0, _K)], sigv.at[b], s_w[b])
        pltpu.async_copy(x24_hbm.at[dstv.at[b]], g24v.at[b], s_g24[b])
        pltpu.async_copy(x3_hbm.at[srcv.at[b]], g3v.at[b], s_g3[b])

    def _wait_gathers(b):
        pltpu.make_async_copy(sig_hbm.at[pl.ds(0, _K)], sigv.at[b], s_w[b]).wait()
        pltpu.make_async_copy(
            x24_hbm.at[pl.ds(0, _K)], g24v.at[b], s_g24[b]).wait()
        pltpu.make_async_copy(
            x3_hbm.at[pl.ds(0, _K)], g3v.at[b], s_g3[b]).wait()

    def _wait_outs(b):
        pltpu.make_async_copy(
            sigv.at[b], aggsh.at[pl.ds(0, _K)], s_sc[b]).wait()
        pltpu.make_async_copy(
            g3v.at[b], g34_hbm.at[pl.ds(0, _K)], s_st[b]).wait()

    def _compute(b):
        def row(j, rcarry):
            for c in range(D // 16):
                sl = pl.ds(c * 16, 16)
                sigv[b, j, sl] = sigv[b, j, sl] * g24v[b, j, sl]
                g3v[b, j, sl] = g3v[b, j, sl] + g24v[b, j, pl.ds(D + c * 16, 16)]
            return rcarry

        lax.fori_loop(0, _K, row, 0, unroll=2)

    def _issue_outs(g, b):
        e0 = base_e + g * _K
        pltpu.async_copy(sigv.at[b], aggsh.at[srcv.at[b]], s_sc[b], add=True)
        pltpu.async_copy(g3v.at[b], g34_hbm.at[pl.ds(e0, _K)], s_st[b])

    _issue_idx(0, 0)
    _issue_idx(1, 1)
    _issue_idx(2, 2)
    _wait_idx(0)
    _issue_gathers(0, 0)
    _wait_idx(1)
    _issue_gathers(1, 1)

    def _slot(g, s):
        b = s & (_NB - 1)
        b2 = (s + 2) & (_NB - 1)
        b3 = (s + 3) & (_NB - 1)

        @pl.when(g < _NCHUNK)
        def _():
            _wait_gathers(b)
            _compute(b)
            _issue_outs(g, b)

        @pl.when((g >= 1) & (g <= _NCHUNK))
        def _():
            _wait_outs(b3)

        @pl.when(g + 3 < _NCHUNK)
        def _():
            _issue_idx(g + 3, b3)

        @pl.when(g + 2 < _NCHUNK)
        def _():
            _wait_idx(b2)
            _issue_gathers(g + 2, b2)

    def body4(i, carry):
        for s in range(_NB):
            _slot(i * _NB + s, s)
        return carry

    lax.fori_loop(0, (_NCHUNK + _NB + 1) // _NB, body4, 0)

    plsc.subcore_barrier()
    for i in range(_RPS // 104):
        r = row0 + i * 104
        pltpu.sync_copy(aggsh.at[pl.ds(r, 104)], aggp_hbm.at[cid, pl.ds(r, 104)])

    @pl.when(sid == _NS - 1)
    def _flush_tail():
        r = _NS * _RPS
        pltpu.sync_copy(aggsh.at[pl.ds(r, 16)], aggp_hbm.at[cid, pl.ds(r, 16)])


_sc_edge_fn = None


def _sc_edge(*args):
    global _sc_edge_fn
    if _sc_edge_fn is None:
        _sc_edge_fn = _make_sc_edge()
    return _sc_edge_fn(*args)


def _make_sc_edge():
    return functools.partial(
        pl.kernel,
        out_type=[
            jax.ShapeDtypeStruct((_E, _D), jnp.float32),
            jax.ShapeDtypeStruct((_NC, _N, _D), jnp.float32),
        ],
        mesh=plsc.VectorSubcoreMesh(
            core_axis_name="c", subcore_axis_name="s", num_cores=_NC),
        scratch_types=[
            pltpu.VMEM((_NB, _K), jnp.int32),
            pltpu.VMEM((_NB, _K), jnp.int32),
            pltpu.VMEM((_NB, _K, _D), jnp.float32),
            pltpu.VMEM((_NB, _K, 2 * _D), jnp.float32),
            pltpu.VMEM((_NB, _K, _D), jnp.float32),
            pltpu.VMEM((8, _D), jnp.float32),
            pltpu.VMEM_SHARED((_N, _D), jnp.float32),
        ] + [pltpu.SemaphoreType.DMA] * (6 * _NB),
    )(_sc_edge_body)


_NC0 = 2
_NW0 = _NC0 * _NS
_EPW0 = _E // _NW0
_K0 = 80
_NCH0 = _EPW0 // _K0
_NB0 = 4


def _sc_dist_body(src_hbm, dst_hbm, lx_hbm, ly_hbm,
                  sq_hbm, cnt_hbm,
                  srcv, dstv, sxv, dxv, syv, dyv, sqv, onesv, zbuf, cntsh,
                  *sems):
    s_ix = sems[0:_NB0]
    s_g = sems[_NB0:2 * _NB0]
    s_sq = sems[2 * _NB0:3 * _NB0]
    s_ct = sems[3 * _NB0:4 * _NB0]
    cid = lax.axis_index("c")
    sid = lax.axis_index("s")
    wid = sid * _NC0 + cid
    base_e = wid * _EPW0
    zero = jnp.zeros((16,), jnp.float32)
    one = jnp.ones((16,), jnp.float32)

    for c in range(_K0 // 16):
        onesv[pl.ds(c * 16, 16)] = one

    def zvec(v, carry):
        zbuf[pl.ds(v * 16, 16)] = zero
        return carry

    lax.fori_loop(0, _RPS // 16, zvec, 0)
    row0 = sid * _RPS
    pltpu.sync_copy(zbuf, cntsh.at[pl.ds(row0, _RPS)])

    @pl.when(sid == _NS - 1)
    def _zero_tail():
        pltpu.sync_copy(zbuf.at[pl.ds(0, 16)], cntsh.at[pl.ds(_NS * _RPS, 16)])

    plsc.subcore_barrier()

    def _issue_idx(g, b):
        e0 = base_e + g * _K0
        pltpu.async_copy(src_hbm.at[pl.ds(e0, _K0)], srcv.at[b], s_ix[b])
        pltpu.async_copy(dst_hbm.at[pl.ds(e0, _K0)], dstv.at[b], s_ix[b])

    def _wait_idx(b):
        pltpu.make_async_copy(src_hbm.at[pl.ds(0, _K0)], srcv.at[b], s_ix[b]).wait()
        pltpu.make_async_copy(dst_hbm.at[pl.ds(0, _K0)], dstv.at[b], s_ix[b]).wait()

    def _issue_gathers(g, b):
        pltpu.async_copy(lx_hbm.at[srcv.at[b]], sxv.at[b], s_g[b])
        pltpu.async_copy(lx_hbm.at[dstv.at[b]], dxv.at[b], s_g[b])
        pltpu.async_copy(ly_hbm.at[srcv.at[b]], syv.at[b], s_g[b])
        pltpu.async_copy(ly_hbm.at[dstv.at[b]], dyv.at[b], s_g[b])

    def _wait_gathers(b):
        pltpu.make_async_copy(lx_hbm.at[pl.ds(0, _K0)], sxv.at[b], s_g[b]).wait()
        pltpu.make_async_copy(lx_hbm.at[pl.ds(0, _K0)], dxv.at[b], s_g[b]).wait()
        pltpu.make_async_copy(ly_hbm.at[pl.ds(0, _K0)], syv.at[b], s_g[b]).wait()
        pltpu.make_async_copy(ly_hbm.at[pl.ds(0, _K0)], dyv.at[b], s_g[b]).wait()

    def _wait_outs(b):
        pltpu.make_async_copy(sqv.at[b], sq_hbm.at[pl.ds(0, _K0)], s_sq[b]).wait()
        pltpu.make_async_copy(onesv, cntsh.at[pl.ds(0, _K0)], s_ct[b]).wait()

    def _compute(b):
        def vec(v, carry):
            sl = pl.ds(v * 16, 16)
            ddx = sxv[b, sl] - dxv[b, sl]
            ddy = syv[b, sl] - dyv[b, sl]
            sqv[b, sl] = ddx * ddx + ddy * ddy + 1e-12
            return carry

        lax.fori_loop(0, _K0 // 16, vec, 0, unroll=2)

    def _issue_outs(g, b):
        e0 = base_e + g * _K0
        pltpu.async_copy(sqv.at[b], sq_hbm.at[pl.ds(e0, _K0)], s_sq[b])
        pltpu.async_copy(onesv, cntsh.at[srcv.at[b]], s_ct[b], add=True)

    _issue_idx(0, 0)
    _issue_idx(1, 1)
    _issue_idx(2, 2)
    _wait_idx(0)
    _issue_gathers(0, 0)
    _wait_idx(1)
    _issue_gathers(1, 1)

    def _slot(g, s):
        b = s & (_NB0 - 1)
        b2 = (s + 2) & (_NB0 - 1)
        b3 = (s + 3) & (_NB0 - 1)

        @pl.when(g < _NCH0)
        def _():
            _wait_gathers(b)
            _compute(b)
            _issue_outs(g, b)

        @pl.when((g >= 1) & (g <= _NCH0))
        def _():
            _wait_outs(b3)

        @pl.when(g + 3 < _NCH0)
        def _():
            _issue_idx(g + 3, b3)

        @pl.when(g + 2 < _NCH0)
        def _():
            _wait_idx(b2)
            _issue_gathers(g + 2, b2)

    def body4(i, carry):
        for s in range(_NB0):
            _slot(i * _NB0 + s, s)
        return carry

    lax.fori_loop(0, (_NCH0 + _NB0 + 1) // _NB0, body4, 0)
    plsc.subcore_barrier()
    cbase = cid * _N
    pltpu.sync_copy(cntsh.at[pl.ds(row0, _RPS)], zbuf)
    pltpu.sync_copy(zbuf, cnt_hbm.at[pl.ds(cbase + row0, _RPS)])

    @pl.when(sid == _NS - 1)
    def _flush_tail():
        r = _NS * _RPS
        pltpu.sync_copy(cntsh.at[pl.ds(r, 16)], zbuf.at[pl.ds(0, 16)])
        pltpu.sync_copy(zbuf.at[pl.ds(0, 16)], cnt_hbm.at[pl.ds(cbase + r, 16)])


_sc_dist_fn = None


def _sc_dist(*args):
    global _sc_dist_fn
    if _sc_dist_fn is None:
        _sc_dist_fn = functools.partial(
            pl.kernel,
            out_type=[
                jax.ShapeDtypeStruct((_E,), jnp.float32),
                jax.ShapeDtypeStruct((_NC0 * _N,), jnp.float32),
            ],
            mesh=plsc.VectorSubcoreMesh(
                core_axis_name="c", subcore_axis_name="s", num_cores=_NC0),
            scratch_types=[
                pltpu.VMEM((_NB0, _K0), jnp.int32),
                pltpu.VMEM((_NB0, _K0), jnp.int32),
                pltpu.VMEM((_NB0, _K0), jnp.float32),
                pltpu.VMEM((_NB0, _K0), jnp.float32),
                pltpu.VMEM((_NB0, _K0), jnp.float32),
                pltpu.VMEM((_NB0, _K0), jnp.float32),
                pltpu.VMEM((_NB0, _K0), jnp.float32),
                pltpu.VMEM((_K0,), jnp.float32),
                pltpu.VMEM((_RPS,), jnp.float32),
                pltpu.VMEM_SHARED((_N,), jnp.float32),
            ] + [pltpu.SemaphoreType.DMA] * (4 * _NB0),
        )(_sc_dist_body)
    return _sc_dist_fn(*args)


def _bn_affine(s, ss, count, gamma, beta):
    mean = s / count
    var = ss / count - mean * mean
    inv = gamma / jnp.sqrt(var + _EPS)
    return inv, beta - mean * inv


def kernel(locs, edge_index, W_init, b_init, W_edge, b_edge, Wv, bv, We, be,
           gamma_v, beta_v, gamma_e, beta_e):
    N, D = locs.shape[0], W_init.shape[1]
    E = edge_index.shape[1]
    L = Wv.shape[0]
    src = edge_index[0]
    dst = edge_index[1]

    locs_t = locs.T
    sq, cnt_p = _sc_dist(src, dst, locs_t[0], locs_t[1])
    cnt_p = cnt_p.reshape(_NC0, N)
    counts = jnp.maximum(cnt_p[0] + cnt_p[1], 1.0).reshape(N, 1)

    dist2 = sq.reshape(E, 1)
    wedge = W_edge.reshape(1, D)
    bedge = b_edge.reshape(1, D)

    node_embed, x1, x24, x3 = _init_tc(locs, W_init, b_init, Wv[0], bv[0], N, D)
    x = node_embed

    w_prev = None
    u_prev = None
    su = ssu = None
    for l in range(L):
        if l == 0:
            w_cur, sig = _edge_tc(
                True, False, True, dist2, wedge, bedge, E=E, D=D)
        elif l == 1:
            A, B = _bn_affine(su, ssu, float(E), gamma_e[l - 1], beta_e[l - 1])
            w_cur, sig = _edge_tc(
                True, True, True,
                dist2, wedge, bedge, u_prev, A.reshape(1, D), B.reshape(1, D),
                E=E, D=D)
        else:
            A, B = _bn_affine(su, ssu, float(E), gamma_e[l - 1], beta_e[l - 1])
            w_cur, sig = _edge_tc(
                False, True, True,
                w_prev, u_prev, A.reshape(1, D), B.reshape(1, D), E=E, D=D)

        g34, aggp = _sc_edge(src, dst, sig, x24, x3)
        t = _matmul_tc(w_cur, We[l], be[l].reshape(1, D), E, D)
        u, st = _stats_tc(t, g34, E, D)
        su, ssu = st[0], st[1]

        lastlayer = l == L - 1
        if lastlayer:
            x = _node_tc(x, x1, aggp, counts, gamma_v[l], beta_v[l],
                         None, None, True, N, D)[0]
        else:
            x, x1, x24, x3 = _node_tc(
                x, x1, aggp, counts, gamma_v[l], beta_v[l],
                Wv[l + 1], bv[l + 1], False, N, D)
        w_prev = w_cur
        u_prev = u

    A, B = _bn_affine(su, ssu, float(E), gamma_e[L - 1], beta_e[L - 1])
    (w_final,) = _edge_tc(
        False, True, False,
        w_prev, u_prev, A.reshape(1, D), B.reshape(1, D), E=E, D=D)
    return (x, w_final, node_embed)


# speedup vs baseline: 4.7287x; 1.5832x over previous
import functools

import jax
import jax.numpy as jnp
from jax import lax
from jax.experimental import pallas as pl
from jax.experimental.pallas import tpu as pltpu
from jax.experimental.pallas import tpu_sc as plsc

_BE = 3200
_EPS = 1e-5

_N = 10000
_E = 320000
_D = 128
_NC = 2
_NS = 16
_NW = _NC * _NS
_EPW = _E // _NW
_K = 16
_NCHUNK = _EPW // _K
_NB = 4
_RPS = 624


def _edge_tc_body(first, has_u, sig_out, *refs):
    i = 0
    if first:
        dist_ref = refs[i]; i += 1
        wedge_ref = refs[i]; i += 1
        bedge_ref = refs[i]; i += 1
    else:
        wprev_ref = refs[i]; i += 1
    if has_u:
        u_ref = refs[i]; i += 1
        a_ref = refs[i]; i += 1
        b_ref = refs[i]; i += 1
    wout_ref = refs[i]; i += 1
    if sig_out:
        sout_ref = refs[i]; i += 1

    if first:
        w_prev = jnp.sqrt(dist_ref[...]) * wedge_ref[...] + bedge_ref[...]
    else:
        w_prev = wprev_ref[...]
    if has_u:
        z = u_ref[...] * a_ref[...] + b_ref[...]
        w = w_prev + z * jax.nn.sigmoid(z)
    else:
        w = w_prev
    wout_ref[...] = w
    if sig_out:
        sout_ref[...] = jax.nn.sigmoid(w)


def _edge_tc(first, has_u, sig_out, *args, E, D):
    grid = (E // _BE,)
    row_spec = pl.BlockSpec((_BE, D), lambda i: (i, 0))
    dist_spec = pl.BlockSpec((_BE, 1), lambda i: (i, 0))
    vec_spec = pl.BlockSpec((1, D), lambda i: (0, 0))
    in_specs = []
    if first:
        in_specs += [dist_spec, vec_spec, vec_spec]
    else:
        in_specs += [row_spec]
    if has_u:
        in_specs += [row_spec, vec_spec, vec_spec]
    n_out = 2 if sig_out else 1
    out_specs = [row_spec] * n_out
    out_shape = [jax.ShapeDtypeStruct((E, D), jnp.float32)] * n_out
    return pl.pallas_call(
        functools.partial(_edge_tc_body, first, has_u, sig_out),
        grid=grid,
        in_specs=in_specs,
        out_specs=out_specs,
        out_shape=out_shape,
    )(*args)


def _matmul_body(w_ref, we_ref, be_ref, t_ref):
    t_ref[...] = (
        jnp.dot(w_ref[...], we_ref[...], preferred_element_type=jnp.float32)
        + be_ref[...]
    )


def _matmul_tc(w, We, be, E, D):
    row_spec = pl.BlockSpec((_BE, D), lambda i: (i, 0))
    return pl.pallas_call(
        _matmul_body,
        grid=(E // _BE,),
        in_specs=[
            row_spec,
            pl.BlockSpec((D, D), lambda i: (0, 0)),
            pl.BlockSpec((1, D), lambda i: (0, 0)),
        ],
        out_specs=row_spec,
        out_shape=jax.ShapeDtypeStruct((E, D), jnp.float32),
    )(w, We, be)


def _stats_body(t_ref, g_ref, u_ref, st_ref):
    u = t_ref[...] + g_ref[...]
    u_ref[...] = u
    blk = jnp.concatenate(
        [jnp.sum(u, axis=0, keepdims=True),
         jnp.sum(u * u, axis=0, keepdims=True)], axis=0)

    @pl.when(pl.program_id(0) == 0)
    def _():
        st_ref[...] = blk

    @pl.when(pl.program_id(0) > 0)
    def _():
        st_ref[...] += blk


def _stats_tc(t, g34, E, D):
    row_spec = pl.BlockSpec((_BE, D), lambda i: (i, 0))
    return pl.pallas_call(
        _stats_body,
        grid=(E // _BE,),
        in_specs=[row_spec, row_spec],
        out_specs=[row_spec, pl.BlockSpec((2, D), lambda i: (0, 0))],
        out_shape=[
            jax.ShapeDtypeStruct((E, D), jnp.float32),
            jax.ShapeDtypeStruct((2, D), jnp.float32),
        ],
    )(t, g34)


def _node_body(nparts, lastlayer, *refs):
    i = 0
    x_ref = refs[i]; i += 1
    x1_ref = refs[i]; i += 1
    aggp_ref = refs[i]; i += 1
    counts_ref = refs[i]; i += 1
    gamma_ref = refs[i]; i += 1
    beta_ref = refs[i]; i += 1
    if not lastlayer:
        wv_ref = refs[i]; i += 1
        bv_ref = refs[i]; i += 1
    xout_ref = refs[i]; i += 1
    if not lastlayer:
        x1o_ref = refs[i]; i += 1
        x24o_ref = refs[i]; i += 1
        x3o_ref = refs[i]; i += 1

    agg = aggp_ref[0]
    for p in range(1, nparts):
        agg = agg + aggp_ref[p]
    agg = agg / counts_ref[...]
    pre = x1_ref[...] + agg
    mean = jnp.mean(pre, axis=0, keepdims=True)
    var = jnp.mean((pre - mean) ** 2, axis=0, keepdims=True)
    xn = gamma_ref[...] * (pre - mean) / jnp.sqrt(var + _EPS) + beta_ref[...]
    x = x_ref[...] + xn * jax.nn.sigmoid(xn)
    xout_ref[...] = x
    if not lastlayer:
        D = x.shape[1]
        mm = [
            jnp.dot(x, wv_ref[k], preferred_element_type=jnp.float32)
            + bv_ref[0, k][None, :]
            for k in range(4)
        ]
        x1o_ref[...] = mm[0]
        x24o_ref[:, :D] = mm[1]
        x24o_ref[:, D:] = mm[3]
        x3o_ref[...] = mm[2]


def _node_tc(x, x1, aggp, counts, gamma, beta, wv, bv, lastlayer, N, D):
    nparts = aggp.shape[0]
    args = [x, x1, aggp, counts, gamma.reshape(1, D), beta.reshape(1, D)]
    in_specs = [
        pl.BlockSpec((N, D), lambda: (0, 0)),
        pl.BlockSpec((N, D), lambda: (0, 0)),
        pl.BlockSpec((nparts, N, D), lambda: (0, 0, 0)),
        pl.BlockSpec((N, 1), lambda: (0, 0)),
        pl.BlockSpec((1, D), lambda: (0, 0)),
        pl.BlockSpec((1, D), lambda: (0, 0)),
    ]
    out_shape = [jax.ShapeDtypeStruct((N, D), jnp.float32)]
    out_specs = [pl.BlockSpec((N, D), lambda: (0, 0))]
    if not lastlayer:
        args += [wv, bv.reshape(1, 4, D)]
        in_specs += [
            pl.BlockSpec((4, D, D), lambda: (0, 0, 0)),
            pl.BlockSpec((1, 4, D), lambda: (0, 0, 0)),
        ]
        out_shape += [
            jax.ShapeDtypeStruct((N, D), jnp.float32),
            jax.ShapeDtypeStruct((N, 2 * D), jnp.float32),
            jax.ShapeDtypeStruct((N, D), jnp.float32),
        ]
        out_specs += [
            pl.BlockSpec((N, D), lambda: (0, 0)),
            pl.BlockSpec((N, 2 * D), lambda: (0, 0)),
            pl.BlockSpec((N, D), lambda: (0, 0)),
        ]
    return pl.pallas_call(
        functools.partial(_node_body, nparts, lastlayer),
        in_specs=in_specs,
        out_specs=out_specs,
        out_shape=out_shape,
    )(*args)


def _init_body(locs_ref, wi_ref, bi_ref, wv_ref, bv_ref,
               ne_ref, x1o_ref, x24o_ref, x3o_ref):
    ne = (
        jnp.dot(locs_ref[...], wi_ref[...], preferred_element_type=jnp.float32)
        + bi_ref[...]
    )
    ne_ref[...] = ne
    D = ne.shape[1]
    mm = [
        jnp.dot(ne, wv_ref[k], preferred_element_type=jnp.float32)
        + bv_ref[0, k][None, :]
        for k in range(4)
    ]
    x1o_ref[...] = mm[0]
    x24o_ref[:, :D] = mm[1]
    x24o_ref[:, D:] = mm[3]
    x3o_ref[...] = mm[2]


def _init_tc(locs, W_init, b_init, wv0, bv0, N, D):
    return pl.pallas_call(
        _init_body,
        in_specs=[
            pl.BlockSpec((N, 2), lambda: (0, 0)),
            pl.BlockSpec((2, D), lambda: (0, 0)),
            pl.BlockSpec((1, D), lambda: (0, 0)),
            pl.BlockSpec((4, D, D), lambda: (0, 0, 0)),
            pl.BlockSpec((1, 4, D), lambda: (0, 0, 0)),
        ],
        out_specs=[
            pl.BlockSpec((N, D), lambda: (0, 0)),
            pl.BlockSpec((N, D), lambda: (0, 0)),
            pl.BlockSpec((N, 2 * D), lambda: (0, 0)),
            pl.BlockSpec((N, D), lambda: (0, 0)),
        ],
        out_shape=[
            jax.ShapeDtypeStruct((N, D), jnp.float32),
            jax.ShapeDtypeStruct((N, D), jnp.float32),
            jax.ShapeDtypeStruct((N, 2 * D), jnp.float32),
            jax.ShapeDtypeStruct((N, D), jnp.float32),
        ],
    )(locs, W_init, b_init.reshape(1, D), wv0, bv0.reshape(1, 4, D))


def _sc_edge_body(src_hbm, dst_hbm, sig_hbm, x24_hbm, x3_hbm,
                  g34_hbm, aggp_hbm,
                  srcv, dstv, sigv, g24v, g3v, zbuf, aggsh, *sems):
    D = _D
    s_ix = sems[0:_NB]
    s_w = sems[_NB:2 * _NB]
    s_g24 = sems[2 * _NB:3 * _NB]
    s_g3 = sems[3 * _NB:4 * _NB]
    s_sc = sems[4 * _NB:5 * _NB]
    s_st = sems[5 * _NB:6 * _NB]
    cid = lax.axis_index("c")
    sid = lax.axis_index("s")
    wid = sid * _NC + cid
    base_e = wid * _EPW
    zero = jnp.zeros((16,), jnp.float32)

    def zrow(j, carry):
        for c in range(D // 16):
            zbuf[j, pl.ds(c * 16, 16)] = zero
        return carry

    lax.fori_loop(0, 8, zrow, 0)
    row0 = sid * _RPS
    for i in range(_RPS // 8):
        pltpu.sync_copy(zbuf, aggsh.at[pl.ds(row0 + i * 8, 8)])

    @pl.when(sid == _NS - 1)
    def _zero_tail():
        pltpu.sync_copy(zbuf, aggsh.at[pl.ds(_NS * _RPS, 8)])
        pltpu.sync_copy(zbuf, aggsh.at[pl.ds(_NS * _RPS + 8, 8)])

    plsc.subcore_barrier()

    def _issue_idx(g, b):
        e0 = base_e + g * _K
        pltpu.async_copy(src_hbm.at[pl.ds(e0, _K)], srcv.at[b], s_ix[b])
        pltpu.async_copy(dst_hbm.at[pl.ds(e0, _K)], dstv.at[b], s_ix[b])

    def _wait_idx(b):
        pltpu.make_async_copy(src_hbm.at[pl.ds(0, _K)], srcv.at[b], s_ix[b]).wait()
        pltpu.make_async_copy(dst_hbm.at[pl.ds(0, _K)], dstv.at[b], s_ix[b]).wait()

    def _issue_gathers(g, b):
        e0 = base_e + g * _K
        pltpu.async_copy(sig_hbm.at[pl.ds(e0, _K)], sigv.at[b], s_w[b])
        pltpu.async_copy(x24_hbm.at[dstv.at[b]], g24v.at[b], s_g24[b])
        pltpu.async_copy(x3_hbm.at[srcv.at[b]], g3v.at[b], s_g3[b])

    def _wait_gathers(b):
        pltpu.make_async_copy(sig_hbm.at[pl.ds(0, _K)], sigv.at[b], s_w[b]).wait()
        pltpu.make_async_copy(
            x24_hbm.at[pl.ds(0, _K)], g24v.at[b], s_g24[b]).wait()
        pltpu.make_async_copy(
            x3_hbm.at[pl.ds(0, _K)], g3v.at[b], s_g3[b]).wait()

    def _wait_outs(b):
        pltpu.make_async_copy(
            sigv.at[b], aggsh.at[pl.ds(0, _K)], s_sc[b]).wait()
        pltpu.make_async_copy(
            g3v.at[b], g34_hbm.at[pl.ds(0, _K)], s_st[b]).wait()

    def _compute(b):
        def row(j, rcarry):
            for c in range(D // 16):
                sl = pl.ds(c * 16, 16)
                sigv[b, j, sl] = sigv[b, j, sl] * g24v[b, j, sl]
                g3v[b, j, sl] = g3v[b, j, sl] + g24v[b, j, pl.ds(D + c * 16, 16)]
            return rcarry

        lax.fori_loop(0, _K, row, 0, unroll=2)

    def _issue_outs(g, b):
        e0 = base_e + g * _K
        pltpu.async_copy(sigv.at[b], aggsh.at[srcv.at[b]], s_sc[b], add=True)
        pltpu.async_copy(g3v.at[b], g34_hbm.at[pl.ds(e0, _K)], s_st[b])

    _issue_idx(0, 0)
    _issue_idx(1, 1)
    _issue_idx(2, 2)
    _wait_idx(0)
    _issue_gathers(0, 0)
    _wait_idx(1)
    _issue_gathers(1, 1)

    def _slot(g, s):
        b = s & (_NB - 1)
        b2 = (s + 2) & (_NB - 1)
        b3 = (s + 3) & (_NB - 1)

        @pl.when(g < _NCHUNK)
        def _():
            _wait_gathers(b)
            _compute(b)
            _issue_outs(g, b)

        @pl.when((g >= 1) & (g <= _NCHUNK))
        def _():
            _wait_outs(b3)

        @pl.when(g + 3 < _NCHUNK)
        def _():
            _issue_idx(g + 3, b3)

        @pl.when(g + 2 < _NCHUNK)
        def _():
            _wait_idx(b2)
            _issue_gathers(g + 2, b2)

    def body4(i, carry):
        for s in range(_NB):
            _slot(i * _NB + s, s)
        return carry

    lax.fori_loop(0, (_NCHUNK + _NB + 1) // _NB, body4, 0)

    plsc.subcore_barrier()
    for i in range(_RPS // 104):
        r = row0 + i * 104
        pltpu.sync_copy(aggsh.at[pl.ds(r, 104)], aggp_hbm.at[cid, pl.ds(r, 104)])

    @pl.when(sid == _NS - 1)
    def _flush_tail():
        r = _NS * _RPS
        pltpu.sync_copy(aggsh.at[pl.ds(r, 16)], aggp_hbm.at[cid, pl.ds(r, 16)])


_sc_edge_fn = None


def _sc_edge(*args):
    global _sc_edge_fn
    if _sc_edge_fn is None:
        _sc_edge_fn = _make_sc_edge()
    return _sc_edge_fn(*args)


def _make_sc_edge():
    return functools.partial(
        pl.kernel,
        out_type=[
            jax.ShapeDtypeStruct((_E, _D), jnp.float32),
            jax.ShapeDtypeStruct((_NC, _N, _D), jnp.float32),
        ],
        mesh=plsc.VectorSubcoreMesh(
            core_axis_name="c", subcore_axis_name="s", num_cores=_NC),
        scratch_types=[
            pltpu.VMEM((_NB, _K), jnp.int32),
            pltpu.VMEM((_NB, _K), jnp.int32),
            pltpu.VMEM((_NB, _K, _D), jnp.float32),
            pltpu.VMEM((_NB, _K, 2 * _D), jnp.float32),
            pltpu.VMEM((_NB, _K, _D), jnp.float32),
            pltpu.VMEM((8, _D), jnp.float32),
            pltpu.VMEM_SHARED((_N, _D), jnp.float32),
        ] + [pltpu.SemaphoreType.DMA] * (6 * _NB),
    )(_sc_edge_body)


_NC0 = 2
_NW0 = _NC0 * _NS
_EPW0 = _E // _NW0
_K0 = 80
_NCH0 = _EPW0 // _K0
_NB0 = 4


def _sc_dist_body(src_hbm, dst_hbm, lx_hbm, ly_hbm,
                  sq_hbm, cnt_hbm,
                  srcv, dstv, sxv, dxv, syv, dyv, sqv, onesv, zbuf, cntsh,
                  *sems):
    s_ix = sems[0:_NB0]
    s_g = sems[_NB0:2 * _NB0]
    s_sq = sems[2 * _NB0:3 * _NB0]
    s_ct = sems[3 * _NB0:4 * _NB0]
    cid = lax.axis_index("c")
    sid = lax.axis_index("s")
    wid = sid * _NC0 + cid
    base_e = wid * _EPW0
    zero = jnp.zeros((16,), jnp.float32)
    one = jnp.ones((16,), jnp.float32)

    for c in range(_K0 // 16):
        onesv[pl.ds(c * 16, 16)] = one

    def zvec(v, carry):
        zbuf[pl.ds(v * 16, 16)] = zero
        return carry

    lax.fori_loop(0, _RPS // 16, zvec, 0)
    row0 = sid * _RPS
    pltpu.sync_copy(zbuf, cntsh.at[pl.ds(row0, _RPS)])

    @pl.when(sid == _NS - 1)
    def _zero_tail():
        pltpu.sync_copy(zbuf.at[pl.ds(0, 16)], cntsh.at[pl.ds(_NS * _RPS, 16)])

    plsc.subcore_barrier()

    def _issue_idx(g, b):
        e0 = base_e + g * _K0
        pltpu.async_copy(src_hbm.at[pl.ds(e0, _K0)], srcv.at[b], s_ix[b])
        pltpu.async_copy(dst_hbm.at[pl.ds(e0, _K0)], dstv.at[b], s_ix[b])

    def _wait_idx(b):
        pltpu.make_async_copy(src_hbm.at[pl.ds(0, _K0)], srcv.at[b], s_ix[b]).wait()
        pltpu.make_async_copy(dst_hbm.at[pl.ds(0, _K0)], dstv.at[b], s_ix[b]).wait()

    def _issue_gathers(g, b):
        pltpu.async_copy(lx_hbm.at[srcv.at[b]], sxv.at[b], s_g[b])
        pltpu.async_copy(lx_hbm.at[dstv.at[b]], dxv.at[b], s_g[b])
        pltpu.async_copy(ly_hbm.at[srcv.at[b]], syv.at[b], s_g[b])
        pltpu.async_copy(ly_hbm.at[dstv.at[b]], dyv.at[b], s_g[b])

    def _wait_gathers(b):
        pltpu.make_async_copy(lx_hbm.at[pl.ds(0, _K0)], sxv.at[b], s_g[b]).wait()
        pltpu.make_async_copy(lx_hbm.at[pl.ds(0, _K0)], dxv.at[b], s_g[b]).wait()
        pltpu.make_async_copy(ly_hbm.at[pl.ds(0, _K0)], syv.at[b], s_g[b]).wait()
        pltpu.make_async_copy(ly_hbm.at[pl.ds(0, _K0)], dyv.at[b], s_g[b]).wait()

    def _wait_outs(b):
        pltpu.make_async_copy(sqv.at[b], sq_hbm.at[pl.ds(0, _K0)], s_sq[b]).wait()
        pltpu.make_async_copy(onesv, cntsh.at[pl.ds(0, _K0)], s_ct[b]).wait()

    def _compute(b):
        def vec(v, carry):
            sl = pl.ds(v * 16, 16)
            ddx = sxv[b, sl] - dxv[b, sl]
            ddy = syv[b, sl] - dyv[b, sl]
            sqv[b, sl] = ddx * ddx + ddy * ddy + 1e-12
            return carry

        lax.fori_loop(0, _K0 // 16, vec, 0, unroll=2)

    def _issue_outs(g, b):
        e0 = base_e + g * _K0
        pltpu.async_copy(sqv.at[b], sq_hbm.at[pl.ds(e0, _K0)], s_sq[b])
        pltpu.async_copy(onesv, cntsh.at[srcv.at[b]], s_ct[b], add=True)

    _issue_idx(0, 0)
    _issue_idx(1, 1)
    _issue_idx(2, 2)
    _wait_idx(0)
    _issue_gathers(0, 0)
    _wait_idx(1)
    _issue_gathers(1, 1)

    def _slot(g, s):
        b = s & (_NB0 - 1)
        b2 = (s + 2) & (_NB0 - 1)
        b3 = (s + 3) & (_NB0 - 1)

        @pl.when(g < _NCH0)
        def _():
            _wait_gathers(b)
            _compute(b)
            _issue_outs(g, b)

        @pl.when((g >= 1) & (g <= _NCH0))
        def _():
            _wait_outs(b3)

        @pl.when(g + 3 < _NCH0)
        def _():
            _issue_idx(g + 3, b3)

        @pl.when(g + 2 < _NCH0)
        def _():
            _wait_idx(b2)
            _issue_gathers(g + 2, b2)

    def body4(i, carry):
        for s in range(_NB0):
            _slot(i * _NB0 + s, s)
        return carry

    lax.fori_loop(0, (_NCH0 + _NB0 + 1) // _NB0, body4, 0)
    plsc.subcore_barrier()
    cbase = cid * _N
    pltpu.sync_copy(cntsh.at[pl.ds(row0, _RPS)], zbuf)
    pltpu.sync_copy(zbuf, cnt_hbm.at[pl.ds(cbase + row0, _RPS)])

    @pl.when(sid == _NS - 1)
    def _flush_tail():
        r = _NS * _RPS
        pltpu.sync_copy(cntsh.at[pl.ds(r, 16)], zbuf.at[pl.ds(0, 16)])
        pltpu.sync_copy(zbuf.at[pl.ds(0, 16)], cnt_hbm.at[pl.ds(cbase + r, 16)])


_sc_dist_fn = None


def _sc_dist(*args):
    global _sc_dist_fn
    if _sc_dist_fn is None:
        _sc_dist_fn = functools.partial(
            pl.kernel,
            out_type=[
                jax.ShapeDtypeStruct((_E,), jnp.float32),
                jax.ShapeDtypeStruct((_NC0 * _N,), jnp.float32),
            ],
            mesh=plsc.VectorSubcoreMesh(
                core_axis_name="c", subcore_axis_name="s", num_cores=_NC0),
            scratch_types=[
                pltpu.VMEM((_NB0, _K0), jnp.int32),
                pltpu.VMEM((_NB0, _K0), jnp.int32),
                pltpu.VMEM((_NB0, _K0), jnp.float32),
                pltpu.VMEM((_NB0, _K0), jnp.float32),
                pltpu.VMEM((_NB0, _K0), jnp.float32),
                pltpu.VMEM((_NB0, _K0), jnp.float32),
                pltpu.VMEM((_NB0, _K0), jnp.float32),
                pltpu.VMEM((_K0,), jnp.float32),
                pltpu.VMEM((_RPS,), jnp.float32),
                pltpu.VMEM_SHARED((_N,), jnp.float32),
            ] + [pltpu.SemaphoreType.DMA] * (4 * _NB0),
        )(_sc_dist_body)
    return _sc_dist_fn(*args)


def _bn_affine(s, ss, count, gamma, beta):
    mean = s / count
    var = ss / count - mean * mean
    inv = gamma / jnp.sqrt(var + _EPS)
    return inv, beta - mean * inv


def kernel(locs, edge_index, W_init, b_init, W_edge, b_edge, Wv, bv, We, be,
           gamma_v, beta_v, gamma_e, beta_e):
    N, D = locs.shape[0], W_init.shape[1]
    E = edge_index.shape[1]
    L = Wv.shape[0]
    src = edge_index[0]
    dst = edge_index[1]

    locs_t = locs.T
    sq, cnt_p = _sc_dist(src, dst, locs_t[0], locs_t[1])
    cnt_p = cnt_p.reshape(_NC0, N)
    counts = jnp.maximum(cnt_p[0] + cnt_p[1], 1.0).reshape(N, 1)

    dist2 = sq.reshape(E, 1)
    wedge = W_edge.reshape(1, D)
    bedge = b_edge.reshape(1, D)

    node_embed, x1, x24, x3 = _init_tc(locs, W_init, b_init, Wv[0], bv[0], N, D)
    x = node_embed

    w_prev = None
    u_prev = None
    su = ssu = None
    for l in range(L):
        if l == 0:
            w_cur, sig = _edge_tc(
                True, False, True, dist2, wedge, bedge, E=E, D=D)
        elif l == 1:
            A, B = _bn_affine(su, ssu, float(E), gamma_e[l - 1], beta_e[l - 1])
            w_cur, sig = _edge_tc(
                True, True, True,
                dist2, wedge, bedge, u_prev, A.reshape(1, D), B.reshape(1, D),
                E=E, D=D)
        else:
            A, B = _bn_affine(su, ssu, float(E), gamma_e[l - 1], beta_e[l - 1])
            w_cur, sig = _edge_tc(
                False, True, True,
                w_prev, u_prev, A.reshape(1, D), B.reshape(1, D), E=E, D=D)

        g34, aggp = _sc_edge(src, dst, sig, x24, x3)
        t = _matmul_tc(w_cur, We[l], be[l].reshape(1, D), E, D)
        u, st = _stats_tc(t, g34, E, D)
        su, ssu = st[0], st[1]

        lastlayer = l == L - 1
        if lastlayer:
            x = _node_tc(x, x1, aggp, counts, gamma_v[l], beta_v[l],
                         None, None, True, N, D)[0]
        else:
            x, x1, x24, x3 = _node_tc(
                x, x1, aggp, counts, gamma_v[l], beta_v[l],
                Wv[l + 1], bv[l + 1], False, N, D)
        w_prev = w_cur
        u_prev = u

    A, B = _bn_affine(su, ssu, float(E), gamma_e[L - 1], beta_e[L - 1])
    (w_final,) = _edge_tc(
        False, True, False,
        w_prev, u_prev, A.reshape(1, D), B.reshape(1, D), E=E, D=D)
    return (x, w_final, node_embed)


# speedup vs baseline: 4.7790x; 1.0106x over previous
import functools

import jax
import jax.numpy as jnp
from jax import lax
from jax.experimental import pallas as pl
from jax.experimental.pallas import tpu as pltpu
from jax.experimental.pallas import tpu_sc as plsc

_BE = 3200
_EPS = 1e-5

_N = 10000
_E = 320000
_D = 128
_NC = 2
_NS = 16
_NW = _NC * _NS
_EPW = _E // _NW
_K = 16
_NCHUNK = _EPW // _K
_NB = 4
_RPS = 624


def _edge_tc_body(first, has_u, sig_out, *refs):
    i = 0
    if first:
        dist_ref = refs[i]; i += 1
        wedge_ref = refs[i]; i += 1
        bedge_ref = refs[i]; i += 1
    else:
        wprev_ref = refs[i]; i += 1
    if has_u:
        u_ref = refs[i]; i += 1
        a_ref = refs[i]; i += 1
        b_ref = refs[i]; i += 1
    wout_ref = refs[i]; i += 1
    if sig_out:
        sout_ref = refs[i]; i += 1

    if first:
        w_prev = jnp.sqrt(dist_ref[...]) * wedge_ref[...] + bedge_ref[...]
    else:
        w_prev = wprev_ref[...]
    if has_u:
        z = u_ref[...] * a_ref[...] + b_ref[...]
        w = w_prev + z * jax.nn.sigmoid(z)
    else:
        w = w_prev
    wout_ref[...] = w
    if sig_out:
        sout_ref[...] = jax.nn.sigmoid(w)


def _edge_tc(first, has_u, sig_out, *args, E, D):
    grid = (E // _BE,)
    row_spec = pl.BlockSpec((_BE, D), lambda i: (i, 0))
    dist_spec = pl.BlockSpec((_BE, 1), lambda i: (i, 0))
    vec_spec = pl.BlockSpec((1, D), lambda i: (0, 0))
    in_specs = []
    if first:
        in_specs += [dist_spec, vec_spec, vec_spec]
    else:
        in_specs += [row_spec]
    if has_u:
        in_specs += [row_spec, vec_spec, vec_spec]
    n_out = 2 if sig_out else 1
    out_specs = [row_spec] * n_out
    out_shape = [jax.ShapeDtypeStruct((E, D), jnp.float32)] * n_out
    return pl.pallas_call(
        functools.partial(_edge_tc_body, first, has_u, sig_out),
        grid=grid,
        in_specs=in_specs,
        out_specs=out_specs,
        out_shape=out_shape,
    )(*args)


def _matmul_body(w_ref, we_ref, be_ref, t_ref):
    t_ref[...] = (
        jnp.dot(w_ref[...], we_ref[...], preferred_element_type=jnp.float32)
        + be_ref[...]
    )


def _matmul_tc(w, We, be, E, D):
    row_spec = pl.BlockSpec((_BE, D), lambda i: (i, 0))
    return pl.pallas_call(
        _matmul_body,
        grid=(E // _BE,),
        in_specs=[
            row_spec,
            pl.BlockSpec((D, D), lambda i: (0, 0)),
            pl.BlockSpec((1, D), lambda i: (0, 0)),
        ],
        out_specs=row_spec,
        out_shape=jax.ShapeDtypeStruct((E, D), jnp.float32),
    )(w, We, be)


def _stats_body(w_ref, g_ref, we_ref, be_ref, u_ref, st_ref):
    u = (
        jnp.dot(w_ref[...], we_ref[...], preferred_element_type=jnp.float32)
        + be_ref[...] + g_ref[...]
    )
    u_ref[...] = u
    blk = jnp.concatenate(
        [jnp.sum(u, axis=0, keepdims=True),
         jnp.sum(u * u, axis=0, keepdims=True)], axis=0)

    @pl.when(pl.program_id(0) == 0)
    def _():
        st_ref[...] = blk

    @pl.when(pl.program_id(0) > 0)
    def _():
        st_ref[...] += blk


def _stats_tc(w, g34, We, be, E, D):
    row_spec = pl.BlockSpec((_BE, D), lambda i: (i, 0))
    return pl.pallas_call(
        _stats_body,
        grid=(E // _BE,),
        in_specs=[
            row_spec, row_spec,
            pl.BlockSpec((D, D), lambda i: (0, 0)),
            pl.BlockSpec((1, D), lambda i: (0, 0)),
        ],
        out_specs=[row_spec, pl.BlockSpec((2, D), lambda i: (0, 0))],
        out_shape=[
            jax.ShapeDtypeStruct((E, D), jnp.float32),
            jax.ShapeDtypeStruct((2, D), jnp.float32),
        ],
    )(w, g34, We, be)


def _node_body(nparts, lastlayer, *refs):
    i = 0
    x_ref = refs[i]; i += 1
    x1_ref = refs[i]; i += 1
    aggp_ref = refs[i]; i += 1
    counts_ref = refs[i]; i += 1
    gamma_ref = refs[i]; i += 1
    beta_ref = refs[i]; i += 1
    if not lastlayer:
        wv_ref = refs[i]; i += 1
        bv_ref = refs[i]; i += 1
    xout_ref = refs[i]; i += 1
    if not lastlayer:
        x1o_ref = refs[i]; i += 1
        x24o_ref = refs[i]; i += 1
        x3o_ref = refs[i]; i += 1

    agg = aggp_ref[0]
    for p in range(1, nparts):
        agg = agg + aggp_ref[p]
    agg = agg / counts_ref[...]
    pre = x1_ref[...] + agg
    mean = jnp.mean(pre, axis=0, keepdims=True)
    var = jnp.mean((pre - mean) ** 2, axis=0, keepdims=True)
    xn = gamma_ref[...] * (pre - mean) / jnp.sqrt(var + _EPS) + beta_ref[...]
    x = x_ref[...] + xn * jax.nn.sigmoid(xn)
    xout_ref[...] = x
    if not lastlayer:
        D = x.shape[1]
        mm = [
            jnp.dot(x, wv_ref[k], preferred_element_type=jnp.float32)
            + bv_ref[0, k][None, :]
            for k in range(4)
        ]
        x1o_ref[...] = mm[0]
        x24o_ref[:, :D] = mm[1]
        x24o_ref[:, D:] = mm[3]
        x3o_ref[...] = mm[2]


def _node_tc(x, x1, aggp, counts, gamma, beta, wv, bv, lastlayer, N, D):
    nparts = aggp.shape[0]
    args = [x, x1, aggp, counts, gamma.reshape(1, D), beta.reshape(1, D)]
    in_specs = [
        pl.BlockSpec((N, D), lambda: (0, 0)),
        pl.BlockSpec((N, D), lambda: (0, 0)),
        pl.BlockSpec((nparts, N, D), lambda: (0, 0, 0)),
        pl.BlockSpec((N, 1), lambda: (0, 0)),
        pl.BlockSpec((1, D), lambda: (0, 0)),
        pl.BlockSpec((1, D), lambda: (0, 0)),
    ]
    out_shape = [jax.ShapeDtypeStruct((N, D), jnp.float32)]
    out_specs = [pl.BlockSpec((N, D), lambda: (0, 0))]
    if not lastlayer:
        args += [wv, bv.reshape(1, 4, D)]
        in_specs += [
            pl.BlockSpec((4, D, D), lambda: (0, 0, 0)),
            pl.BlockSpec((1, 4, D), lambda: (0, 0, 0)),
        ]
        out_shape += [
            jax.ShapeDtypeStruct((N, D), jnp.float32),
            jax.ShapeDtypeStruct((N, 2 * D), jnp.float32),
            jax.ShapeDtypeStruct((N, D), jnp.float32),
        ]
        out_specs += [
            pl.BlockSpec((N, D), lambda: (0, 0)),
            pl.BlockSpec((N, 2 * D), lambda: (0, 0)),
            pl.BlockSpec((N, D), lambda: (0, 0)),
        ]
    return pl.pallas_call(
        functools.partial(_node_body, nparts, lastlayer),
        in_specs=in_specs,
        out_specs=out_specs,
        out_shape=out_shape,
    )(*args)


def _init_body(locs_ref, wi_ref, bi_ref, wv_ref, bv_ref,
               ne_ref, x1o_ref, x24o_ref, x3o_ref):
    ne = (
        jnp.dot(locs_ref[...], wi_ref[...], preferred_element_type=jnp.float32)
        + bi_ref[...]
    )
    ne_ref[...] = ne
    D = ne.shape[1]
    mm = [
        jnp.dot(ne, wv_ref[k], preferred_element_type=jnp.float32)
        + bv_ref[0, k][None, :]
        for k in range(4)
    ]
    x1o_ref[...] = mm[0]
    x24o_ref[:, :D] = mm[1]
    x24o_ref[:, D:] = mm[3]
    x3o_ref[...] = mm[2]


def _init_tc(locs, W_init, b_init, wv0, bv0, N, D):
    return pl.pallas_call(
        _init_body,
        in_specs=[
            pl.BlockSpec((N, 2), lambda: (0, 0)),
            pl.BlockSpec((2, D), lambda: (0, 0)),
            pl.BlockSpec((1, D), lambda: (0, 0)),
            pl.BlockSpec((4, D, D), lambda: (0, 0, 0)),
            pl.BlockSpec((1, 4, D), lambda: (0, 0, 0)),
        ],
        out_specs=[
            pl.BlockSpec((N, D), lambda: (0, 0)),
            pl.BlockSpec((N, D), lambda: (0, 0)),
            pl.BlockSpec((N, 2 * D), lambda: (0, 0)),
            pl.BlockSpec((N, D), lambda: (0, 0)),
        ],
        out_shape=[
            jax.ShapeDtypeStruct((N, D), jnp.float32),
            jax.ShapeDtypeStruct((N, D), jnp.float32),
            jax.ShapeDtypeStruct((N, 2 * D), jnp.float32),
            jax.ShapeDtypeStruct((N, D), jnp.float32),
        ],
    )(locs, W_init, b_init.reshape(1, D), wv0, bv0.reshape(1, 4, D))


def _sc_edge_body(src_hbm, dst_hbm, sig_hbm, x24_hbm, x3_hbm,
                  g34_hbm, aggp_hbm,
                  srcv, dstv, sigv, g24v, g3v, zbuf, aggsh, *sems):
    D = _D
    s_ix = sems[0:_NB]
    s_w = sems[_NB:2 * _NB]
    s_g24 = sems[2 * _NB:3 * _NB]
    s_g3 = sems[3 * _NB:4 * _NB]
    s_sc = sems[4 * _NB:5 * _NB]
    s_st = sems[5 * _NB:6 * _NB]
    cid = lax.axis_index("c")
    sid = lax.axis_index("s")
    wid = sid * _NC + cid
    base_e = wid * _EPW
    zero = jnp.zeros((16,), jnp.float32)

    def zrow(j, carry):
        for c in range(D // 16):
            zbuf[j, pl.ds(c * 16, 16)] = zero
        return carry

    lax.fori_loop(0, 8, zrow, 0)
    row0 = sid * _RPS
    for i in range(_RPS // 8):
        pltpu.sync_copy(zbuf, aggsh.at[pl.ds(row0 + i * 8, 8)])

    @pl.when(sid == _NS - 1)
    def _zero_tail():
        pltpu.sync_copy(zbuf, aggsh.at[pl.ds(_NS * _RPS, 8)])
        pltpu.sync_copy(zbuf, aggsh.at[pl.ds(_NS * _RPS + 8, 8)])

    plsc.subcore_barrier()

    def _issue_idx(g, b):
        e0 = base_e + g * _K
        pltpu.async_copy(src_hbm.at[pl.ds(e0, _K)], srcv.at[b], s_ix[b])
        pltpu.async_copy(dst_hbm.at[pl.ds(e0, _K)], dstv.at[b], s_ix[b])

    def _wait_idx(b):
        pltpu.make_async_copy(src_hbm.at[pl.ds(0, _K)], srcv.at[b], s_ix[b]).wait()
        pltpu.make_async_copy(dst_hbm.at[pl.ds(0, _K)], dstv.at[b], s_ix[b]).wait()

    def _issue_gathers(g, b):
        e0 = base_e + g * _K
        pltpu.async_copy(sig_hbm.at[pl.ds(e0, _K)], sigv.at[b], s_w[b])
        pltpu.async_copy(x24_hbm.at[dstv.at[b]], g24v.at[b], s_g24[b])
        pltpu.async_copy(x3_hbm.at[srcv.at[b]], g3v.at[b], s_g3[b])

    def _wait_gathers(b):
        pltpu.make_async_copy(sig_hbm.at[pl.ds(0, _K)], sigv.at[b], s_w[b]).wait()
        pltpu.make_async_copy(
            x24_hbm.at[pl.ds(0, _K)], g24v.at[b], s_g24[b]).wait()
        pltpu.make_async_copy(
            x3_hbm.at[pl.ds(0, _K)], g3v.at[b], s_g3[b]).wait()

    def _wait_outs(b):
        pltpu.make_async_copy(
            sigv.at[b], aggsh.at[pl.ds(0, _K)], s_sc[b]).wait()
        pltpu.make_async_copy(
            g3v.at[b], g34_hbm.at[pl.ds(0, _K)], s_st[b]).wait()

    def _compute(b):
        def row(j, rcarry):
            for c in range(D // 16):
                sl = pl.ds(c * 16, 16)
                sigv[b, j, sl] = sigv[b, j, sl] * g24v[b, j, sl]
                g3v[b, j, sl] = g3v[b, j, sl] + g24v[b, j, pl.ds(D + c * 16, 16)]
            return rcarry

        lax.fori_loop(0, _K, row, 0, unroll=2)

    def _issue_outs(g, b):
        e0 = base_e + g * _K
        pltpu.async_copy(sigv.at[b], aggsh.at[srcv.at[b]], s_sc[b], add=True)
        pltpu.async_copy(g3v.at[b], g34_hbm.at[pl.ds(e0, _K)], s_st[b])

    _issue_idx(0, 0)
    _issue_idx(1, 1)
    _issue_idx(2, 2)
    _wait_idx(0)
    _issue_gathers(0, 0)
    _wait_idx(1)
    _issue_gathers(1, 1)

    def _slot(g, s):
        b = s & (_NB - 1)
        b2 = (s + 2) & (_NB - 1)
        b3 = (s + 3) & (_NB - 1)

        @pl.when(g < _NCHUNK)
        def _():
            _wait_gathers(b)
            _compute(b)
            _issue_outs(g, b)

        @pl.when((g >= 1) & (g <= _NCHUNK))
        def _():
            _wait_outs(b3)

        @pl.when(g + 3 < _NCHUNK)
        def _():
            _issue_idx(g + 3, b3)

        @pl.when(g + 2 < _NCHUNK)
        def _():
            _wait_idx(b2)
            _issue_gathers(g + 2, b2)

    def body4(i, carry):
        for s in range(_NB):
            _slot(i * _NB + s, s)
        return carry

    lax.fori_loop(0, (_NCHUNK + _NB + 1) // _NB, body4, 0)

    plsc.subcore_barrier()
    for i in range(_RPS // 104):
        r = row0 + i * 104
        pltpu.sync_copy(aggsh.at[pl.ds(r, 104)], aggp_hbm.at[cid, pl.ds(r, 104)])

    @pl.when(sid == _NS - 1)
    def _flush_tail():
        r = _NS * _RPS
        pltpu.sync_copy(aggsh.at[pl.ds(r, 16)], aggp_hbm.at[cid, pl.ds(r, 16)])


_sc_edge_fn = None


def _sc_edge(*args):
    global _sc_edge_fn
    if _sc_edge_fn is None:
        _sc_edge_fn = _make_sc_edge()
    return _sc_edge_fn(*args)


def _make_sc_edge():
    return functools.partial(
        pl.kernel,
        out_type=[
            jax.ShapeDtypeStruct((_E, _D), jnp.float32),
            jax.ShapeDtypeStruct((_NC, _N, _D), jnp.float32),
        ],
        mesh=plsc.VectorSubcoreMesh(
            core_axis_name="c", subcore_axis_name="s", num_cores=_NC),
        scratch_types=[
            pltpu.VMEM((_NB, _K), jnp.int32),
            pltpu.VMEM((_NB, _K), jnp.int32),
            pltpu.VMEM((_NB, _K, _D), jnp.float32),
            pltpu.VMEM((_NB, _K, 2 * _D), jnp.float32),
            pltpu.VMEM((_NB, _K, _D), jnp.float32),
            pltpu.VMEM((8, _D), jnp.float32),
            pltpu.VMEM_SHARED((_N, _D), jnp.float32),
        ] + [pltpu.SemaphoreType.DMA] * (6 * _NB),
    )(_sc_edge_body)


_NC0 = 2
_NW0 = _NC0 * _NS
_EPW0 = _E // _NW0
_K0 = 80
_NCH0 = _EPW0 // _K0
_NB0 = 4


def _sc_dist_body(src_hbm, dst_hbm, lx_hbm, ly_hbm,
                  sq_hbm, cnt_hbm,
                  srcv, dstv, sxv, dxv, syv, dyv, sqv, onesv, zbuf, cntsh,
                  *sems):
    s_ix = sems[0:_NB0]
    s_g = sems[_NB0:2 * _NB0]
    s_sq = sems[2 * _NB0:3 * _NB0]
    s_ct = sems[3 * _NB0:4 * _NB0]
    cid = lax.axis_index("c")
    sid = lax.axis_index("s")
    wid = sid * _NC0 + cid
    base_e = wid * _EPW0
    zero = jnp.zeros((16,), jnp.float32)
    one = jnp.ones((16,), jnp.float32)

    for c in range(_K0 // 16):
        onesv[pl.ds(c * 16, 16)] = one

    def zvec(v, carry):
        zbuf[pl.ds(v * 16, 16)] = zero
        return carry

    lax.fori_loop(0, _RPS // 16, zvec, 0)
    row0 = sid * _RPS
    pltpu.sync_copy(zbuf, cntsh.at[pl.ds(row0, _RPS)])

    @pl.when(sid == _NS - 1)
    def _zero_tail():
        pltpu.sync_copy(zbuf.at[pl.ds(0, 16)], cntsh.at[pl.ds(_NS * _RPS, 16)])

    plsc.subcore_barrier()

    def _issue_idx(g, b):
        e0 = base_e + g * _K0
        pltpu.async_copy(src_hbm.at[pl.ds(e0, _K0)], srcv.at[b], s_ix[b])
        pltpu.async_copy(dst_hbm.at[pl.ds(e0, _K0)], dstv.at[b], s_ix[b])

    def _wait_idx(b):
        pltpu.make_async_copy(src_hbm.at[pl.ds(0, _K0)], srcv.at[b], s_ix[b]).wait()
        pltpu.make_async_copy(dst_hbm.at[pl.ds(0, _K0)], dstv.at[b], s_ix[b]).wait()

    def _issue_gathers(g, b):
        pltpu.async_copy(lx_hbm.at[srcv.at[b]], sxv.at[b], s_g[b])
        pltpu.async_copy(lx_hbm.at[dstv.at[b]], dxv.at[b], s_g[b])
        pltpu.async_copy(ly_hbm.at[srcv.at[b]], syv.at[b], s_g[b])
        pltpu.async_copy(ly_hbm.at[dstv.at[b]], dyv.at[b], s_g[b])

    def _wait_gathers(b):
        pltpu.make_async_copy(lx_hbm.at[pl.ds(0, _K0)], sxv.at[b], s_g[b]).wait()
        pltpu.make_async_copy(lx_hbm.at[pl.ds(0, _K0)], dxv.at[b], s_g[b]).wait()
        pltpu.make_async_copy(ly_hbm.at[pl.ds(0, _K0)], syv.at[b], s_g[b]).wait()
        pltpu.make_async_copy(ly_hbm.at[pl.ds(0, _K0)], dyv.at[b], s_g[b]).wait()

    def _wait_outs(b):
        pltpu.make_async_copy(sqv.at[b], sq_hbm.at[pl.ds(0, _K0)], s_sq[b]).wait()
        pltpu.make_async_copy(onesv, cntsh.at[pl.ds(0, _K0)], s_ct[b]).wait()

    def _compute(b):
        def vec(v, carry):
            sl = pl.ds(v * 16, 16)
            ddx = sxv[b, sl] - dxv[b, sl]
            ddy = syv[b, sl] - dyv[b, sl]
            sqv[b, sl] = ddx * ddx + ddy * ddy + 1e-12
            return carry

        lax.fori_loop(0, _K0 // 16, vec, 0, unroll=2)

    def _issue_outs(g, b):
        e0 = base_e + g * _K0
        pltpu.async_copy(sqv.at[b], sq_hbm.at[pl.ds(e0, _K0)], s_sq[b])
        pltpu.async_copy(onesv, cntsh.at[srcv.at[b]], s_ct[b], add=True)

    _issue_idx(0, 0)
    _issue_idx(1, 1)
    _issue_idx(2, 2)
    _wait_idx(0)
    _issue_gathers(0, 0)
    _wait_idx(1)
    _issue_gathers(1, 1)

    def _slot(g, s):
        b = s & (_NB0 - 1)
        b2 = (s + 2) & (_NB0 - 1)
        b3 = (s + 3) & (_NB0 - 1)

        @pl.when(g < _NCH0)
        def _():
            _wait_gathers(b)
            _compute(b)
            _issue_outs(g, b)

        @pl.when((g >= 1) & (g <= _NCH0))
        def _():
            _wait_outs(b3)

        @pl.when(g + 3 < _NCH0)
        def _():
            _issue_idx(g + 3, b3)

        @pl.when(g + 2 < _NCH0)
        def _():
            _wait_idx(b2)
            _issue_gathers(g + 2, b2)

    def body4(i, carry):
        for s in range(_NB0):
            _slot(i * _NB0 + s, s)
        return carry

    lax.fori_loop(0, (_NCH0 + _NB0 + 1) // _NB0, body4, 0)
    plsc.subcore_barrier()
    cbase = cid * _N
    pltpu.sync_copy(cntsh.at[pl.ds(row0, _RPS)], zbuf)
    pltpu.sync_copy(zbuf, cnt_hbm.at[pl.ds(cbase + row0, _RPS)])

    @pl.when(sid == _NS - 1)
    def _flush_tail():
        r = _NS * _RPS
        pltpu.sync_copy(cntsh.at[pl.ds(r, 16)], zbuf.at[pl.ds(0, 16)])
        pltpu.sync_copy(zbuf.at[pl.ds(0, 16)], cnt_hbm.at[pl.ds(cbase + r, 16)])


_sc_dist_fn = None


def _sc_dist(*args):
    global _sc_dist_fn
    if _sc_dist_fn is None:
        _sc_dist_fn = functools.partial(
            pl.kernel,
            out_type=[
                jax.ShapeDtypeStruct((_E,), jnp.float32),
                jax.ShapeDtypeStruct((_NC0 * _N,), jnp.float32),
            ],
            mesh=plsc.VectorSubcoreMesh(
                core_axis_name="c", subcore_axis_name="s", num_cores=_NC0),
            scratch_types=[
                pltpu.VMEM((_NB0, _K0), jnp.int32),
                pltpu.VMEM((_NB0, _K0), jnp.int32),
                pltpu.VMEM((_NB0, _K0), jnp.float32),
                pltpu.VMEM((_NB0, _K0), jnp.float32),
                pltpu.VMEM((_NB0, _K0), jnp.float32),
                pltpu.VMEM((_NB0, _K0), jnp.float32),
                pltpu.VMEM((_NB0, _K0), jnp.float32),
                pltpu.VMEM((_K0,), jnp.float32),
                pltpu.VMEM((_RPS,), jnp.float32),
                pltpu.VMEM_SHARED((_N,), jnp.float32),
            ] + [pltpu.SemaphoreType.DMA] * (4 * _NB0),
        )(_sc_dist_body)
    return _sc_dist_fn(*args)


def _bn_affine(s, ss, count, gamma, beta):
    mean = s / count
    var = ss / count - mean * mean
    inv = gamma / jnp.sqrt(var + _EPS)
    return inv, beta - mean * inv


def kernel(locs, edge_index, W_init, b_init, W_edge, b_edge, Wv, bv, We, be,
           gamma_v, beta_v, gamma_e, beta_e):
    N, D = locs.shape[0], W_init.shape[1]
    E = edge_index.shape[1]
    L = Wv.shape[0]
    src = edge_index[0]
    dst = edge_index[1]

    locs_t = locs.T
    sq, cnt_p = _sc_dist(src, dst, locs_t[0], locs_t[1])
    cnt_p = cnt_p.reshape(_NC0, N)
    counts = jnp.maximum(cnt_p[0] + cnt_p[1], 1.0).reshape(N, 1)

    dist2 = sq.reshape(E, 1)
    wedge = W_edge.reshape(1, D)
    bedge = b_edge.reshape(1, D)

    node_embed, x1, x24, x3 = _init_tc(locs, W_init, b_init, Wv[0], bv[0], N, D)
    x = node_embed

    w_prev = None
    u_prev = None
    su = ssu = None
    for l in range(L):
        if l == 0:
            w_cur, sig = _edge_tc(
                True, False, True, dist2, wedge, bedge, E=E, D=D)
        elif l == 1:
            A, B = _bn_affine(su, ssu, float(E), gamma_e[l - 1], beta_e[l - 1])
            w_cur, sig = _edge_tc(
                True, True, True,
                dist2, wedge, bedge, u_prev, A.reshape(1, D), B.reshape(1, D),
                E=E, D=D)
        else:
            A, B = _bn_affine(su, ssu, float(E), gamma_e[l - 1], beta_e[l - 1])
            w_cur, sig = _edge_tc(
                False, True, True,
                w_prev, u_prev, A.reshape(1, D), B.reshape(1, D), E=E, D=D)

        g34, aggp = _sc_edge(src, dst, sig, x24, x3)
        u, st = _stats_tc(w_cur, g34, We[l], be[l].reshape(1, D), E, D)
        su, ssu = st[0], st[1]

        lastlayer = l == L - 1
        if lastlayer:
            x = _node_tc(x, x1, aggp, counts, gamma_v[l], beta_v[l],
                         None, None, True, N, D)[0]
        else:
            x, x1, x24, x3 = _node_tc(
                x, x1, aggp, counts, gamma_v[l], beta_v[l],
                Wv[l + 1], bv[l + 1], False, N, D)
        w_prev = w_cur
        u_prev = u

    A, B = _bn_affine(su, ssu, float(E), gamma_e[L - 1], beta_e[L - 1])
    (w_final,) = _edge_tc(
        False, True, False,
        w_prev, u_prev, A.reshape(1, D), B.reshape(1, D), E=E, D=D)
    return (x, w_final, node_embed)
